# baseline (speedup 1.0000x reference)
import functools

import jax
import jax.numpy as jnp
from jax import lax
from jax.experimental import pallas as pl
from jax.experimental.pallas import tpu as pltpu
from jax.experimental.pallas import tpu_sc as plsc

N = 10000
E = 320000
D = 128
H = 256
NUM_EXPERTS = 64
TOP_K = 2

NC = 2
NS = 16
LANES = 16
NW = NC * NS

EDGES_PER_TILE = E // NW
EDGE_BATCH = 80
NPAD = 10240
ROWS_PER_TILE = NPAD // NS
ZROWS = 128


def _mesh():
  return plsc.VectorSubcoreMesh(
      core_axis_name="c", subcore_axis_name="s", num_cores=NC,
      num_subcores=NS)


def _degrees_body(src_hbm, dst_hbm, out, sidx, didx, ones_v, zbuf,
                  cnt_out_sh, cnt_in_sh):
  c = lax.axis_index("c")
  s = lax.axis_index("s")

  zeros16 = jnp.zeros((LANES,), jnp.float32)

  def zero_body(i, _):
    zbuf[pl.ds(i * LANES, LANES)] = zeros16
    return 0

  lax.fori_loop(0, N // LANES, zero_body, 0)

  @pl.when(s == 0)
  def _():
    pltpu.sync_copy(zbuf, cnt_out_sh)

  @pl.when(s == 1)
  def _():
    pltpu.sync_copy(zbuf, cnt_in_sh)

  def ones_body(i, _):
    ones_v[pl.ds(i * LANES, LANES)] = jnp.ones((LANES,), jnp.float32)
    return 0

  lax.fori_loop(0, EDGE_BATCH // LANES, ones_body, 0)
  plsc.subcore_barrier()

  def count_body(i, _):
    base = c * (E // NC) + s * EDGES_PER_TILE + i * EDGE_BATCH
    pltpu.sync_copy(src_hbm.at[pl.ds(base, EDGE_BATCH)], sidx)
    pltpu.sync_copy(dst_hbm.at[pl.ds(base, EDGE_BATCH)], didx)
    pltpu.sync_copy(ones_v, cnt_out_sh.at[sidx], add=True)
    pltpu.sync_copy(ones_v, cnt_in_sh.at[didx], add=True)
    return 0

  lax.fori_loop(0, EDGES_PER_TILE // EDGE_BATCH, count_body, 0)
  plsc.subcore_barrier()

  @pl.when(s == 0)
  def _():
    pltpu.sync_copy(cnt_out_sh, out.at[c, 0])

  @pl.when(s == 1)
  def _():
    pltpu.sync_copy(cnt_in_sh, out.at[c, 1])


def _degrees(src, dst):
  k = pl.kernel(
      _degrees_body,
      out_type=jax.ShapeDtypeStruct((NC, 2, N), jnp.float32),
      mesh=_mesh(),
      scratch_types=[
          pltpu.VMEM((EDGE_BATCH,), jnp.int32),
          pltpu.VMEM((EDGE_BATCH,), jnp.int32),
          pltpu.VMEM((EDGE_BATCH,), jnp.float32),
          pltpu.VMEM((N,), jnp.float32),
          pltpu.MemorySpace.VMEM_SHARED((N,), jnp.float32),
          pltpu.MemorySpace.VMEM_SHARED((N,), jnp.float32),
      ],
  )
  return k(src, dst)


def _scale_body(cnts_ref, feats_ref, hpre_ref, isr_in_ref):
  cnts = jnp.sum(cnts_ref[...], axis=0)
  deg = jnp.maximum(cnts, 1.0)
  isr = lax.rsqrt(deg)
  hpre_ref[...] = feats_ref[...] * isr[0][:, None]
  isr_in_ref[...] = isr[1][:, None]


def _scale(cnts, feats):
  return pl.pallas_call(
      _scale_body,
      out_shape=(
          jax.ShapeDtypeStruct((N, D), jnp.float32),
          jax.ShapeDtypeStruct((N, 1), jnp.float32),
      ),
  )(cnts, feats)


def _agg_body(hpre, src_hbm, dst_hbm, out, sidx, didx, rows, zbuf, shared,
              sem):
  c = lax.axis_index("c")
  s = lax.axis_index("s")

  zeros16 = jnp.zeros((LANES,), jnp.float32)

  def zero_body(i, _):
    def zcol(j, _):
      zbuf[i, pl.ds(j * LANES, LANES)] = zeros16
      return 0
    lax.fori_loop(0, D // LANES, zcol, 0)
    return 0

  lax.fori_loop(0, ZROWS, zero_body, 0)
  for j in range(ROWS_PER_TILE // ZROWS):
    pltpu.sync_copy(zbuf, shared.at[pl.ds(s * ROWS_PER_TILE + j * ZROWS, ZROWS)])
  plsc.subcore_barrier()

  def edge_body(i, _):
    base = c * (E // NC) + s * EDGES_PER_TILE + i * EDGE_BATCH
    pltpu.sync_copy(src_hbm.at[pl.ds(base, EDGE_BATCH)], sidx)
    pltpu.sync_copy(dst_hbm.at[pl.ds(base, EDGE_BATCH)], didx)
    pltpu.async_copy(hpre.at[sidx], rows, sem).wait()
    pltpu.sync_copy(rows, shared.at[didx], add=True)
    return 0

  lax.fori_loop(0, EDGES_PER_TILE // EDGE_BATCH, edge_body, 0)
  plsc.subcore_barrier()

  for j in range(ROWS_PER_TILE // ZROWS):
    r0 = s * ROWS_PER_TILE + j * ZROWS
    pltpu.sync_copy(shared.at[pl.ds(r0, ZROWS)], out.at[c, pl.ds(r0, ZROWS)])


def _aggregate(hpre, src, dst):
  k = pl.kernel(
      _agg_body,
      out_type=jax.ShapeDtypeStruct((NC, NPAD, D), jnp.float32),
      mesh=_mesh(),
      scratch_types=[
          pltpu.VMEM((EDGE_BATCH,), jnp.int32),
          pltpu.VMEM((EDGE_BATCH,), jnp.int32),
          pltpu.VMEM((EDGE_BATCH, D), jnp.float32),
          pltpu.VMEM((ZROWS, D), jnp.float32),
          pltpu.MemorySpace.VMEM_SHARED((NPAD, D), jnp.float32),
          pltpu.SemaphoreType.DMA,
      ],
  )
  return k(hpre, src, dst)


def _conv_gate_body(p_ref, isr_ref, wc_ref, bc_ref, gw_ref, gb_ref,
                    h_ref, idx_ref, g_ref):
  agg = (p_ref[0, 0:N] + p_ref[1, 0:N]) * isr_ref[...]
  h = jnp.dot(agg, wc_ref[...], preferred_element_type=jnp.float32)
  h = h + bc_ref[...][None, :]
  h_ref[...] = h
  logits = jnp.dot(h, gw_ref[...], preferred_element_type=jnp.float32)
  logits = logits + gb_ref[...][None, :]
  iota = lax.broadcasted_iota(jnp.int32, (N, NUM_EXPERTS), 1)
  m1 = jnp.max(logits, axis=1, keepdims=True)
  i1 = jnp.min(jnp.where(logits == m1, iota, NUM_EXPERTS), axis=1,
               keepdims=True)
  masked = jnp.where(iota == i1, -jnp.inf, logits)
  m2 = jnp.max(masked, axis=1, keepdims=True)
  i2 = jnp.min(jnp.where(masked == m2, iota, NUM_EXPERTS), axis=1,
               keepdims=True)
  e2 = jnp.exp(m2 - m1)
  denom = 1.0 + e2
  g1 = 1.0 / denom
  g2 = e2 / denom
  idx_ref[...] = jnp.concatenate([i1, i2], axis=1)
  g_ref[...] = jnp.concatenate([g1, g2], axis=1)


def _conv_gate(parts, isr_in, W_conv, b_conv, gate_W, gate_b):
  return pl.pallas_call(
      _conv_gate_body,
      out_shape=(
          jax.ShapeDtypeStruct((N, D), jnp.float32),
          jax.ShapeDtypeStruct((N, TOP_K), jnp.int32),
          jax.ShapeDtypeStruct((N, TOP_K), jnp.float32),
      ),
  )(parts, isr_in, W_conv, b_conv, gate_W, gate_b)


A2 = N * TOP_K
BB = 256
PB = 36864
NB = PB // BB
GB = 128
GROWS = PB // NW
NPAD2 = 10240
CTOK = NPAD2 // NW
CB = 40


def _gather_rows_body(h_hbm, tok_hbm, out, tidx, rows, sem):
  c = lax.axis_index("c")
  s = lax.axis_index("s")
  w = s * NC + c
  base = w * GROWS

  def body(i, _):
    r0 = base + i * GB
    pltpu.sync_copy(tok_hbm.at[pl.ds(r0, GB)], tidx)
    pltpu.async_copy(h_hbm.at[tidx], rows, sem).wait()
    pltpu.sync_copy(rows, out.at[pl.ds(r0, GB)])
    return 0

  lax.fori_loop(0, GROWS // GB, body, 0)


def _gather_rows(h, tok_padded):
  k = pl.kernel(
      _gather_rows_body,
      out_type=jax.ShapeDtypeStruct((PB, D), jnp.float32),
      mesh=_mesh(),
      scratch_types=[
          pltpu.VMEM((GB,), jnp.int32),
          pltpu.VMEM((GB, D), jnp.float32),
          pltpu.SemaphoreType.DMA,
      ],
  )
  return k(h, tok_padded)


def _ffn_body(be_ref, x_ref, g_ref, w1_ref, b1_ref, w2_ref, b2_ref, y_ref):
  x = x_ref[...]
  a = jnp.dot(x, w1_ref[0], preferred_element_type=jnp.float32)
  a = a + b1_ref[0]
  a = 0.5 * a * (1.0 + lax.erf(a * 0.7071067811865476))
  y = jnp.dot(a, w2_ref[0], preferred_element_type=jnp.float32)
  y = y + b2_ref[0]
  y_ref[...] = y * g_ref[...]


def _ffn(x_sorted, gate_padded, block_expert, W1, b1, W2, b2):
  grid_spec = pltpu.PrefetchScalarGridSpec(
      num_scalar_prefetch=1,
      grid=(NB,),
      in_specs=[
          pl.BlockSpec((BB, D), lambda b, be: (b, 0)),
          pl.BlockSpec((BB, 1), lambda b, be: (b, 0)),
          pl.BlockSpec((1, D, H), lambda b, be: (be[b], 0, 0)),
          pl.BlockSpec((1, 1, H), lambda b, be: (be[b], 0, 0)),
          pl.BlockSpec((1, H, D), lambda b, be: (be[b], 0, 0)),
          pl.BlockSpec((1, 1, D), lambda b, be: (be[b], 0, 0)),
      ],
      out_specs=pl.BlockSpec((BB, D), lambda b, be: (b, 0)),
  )
  return pl.pallas_call(
      _ffn_body,
      grid_spec=grid_spec,
      out_shape=jax.ShapeDtypeStruct((PB, D), jnp.float32),
  )(block_expert, x_sorted, gate_padded.reshape(PB, 1),
    W1, b1.reshape(NUM_EXPERTS, 1, H), W2, b2.reshape(NUM_EXPERTS, 1, D))


def _combine_body(y_hbm, tok_hbm, out, tidx, rows, zbuf, shared, sem):
  c = lax.axis_index("c")
  s = lax.axis_index("s")
  w = s * NC + c

  zeros16 = jnp.zeros((LANES,), jnp.float32)

  def zero_body(i, _):
    def zcol(j, _):
      zbuf[i, pl.ds(j * LANES, LANES)] = zeros16
      return 0
    lax.fori_loop(0, D // LANES, zcol, 0)
    return 0

  lax.fori_loop(0, ZROWS, zero_body, 0)
  stripe = NPAD2 // NS
  for j in range(stripe // ZROWS):
    pltpu.sync_copy(zbuf, shared.at[pl.ds(s * stripe + j * ZROWS, ZROWS)])
  plsc.subcore_barrier()

  def body(i, _):
    r0 = w * GROWS + i * GB
    pltpu.sync_copy(y_hbm.at[pl.ds(r0, GB)], rows)
    pltpu.sync_copy(tok_hbm.at[pl.ds(r0, GB)], tidx)
    pltpu.sync_copy(rows, shared.at[tidx], add=True)
    return 0

  lax.fori_loop(0, GROWS // GB, body, 0)
  plsc.subcore_barrier()

  for j in range(stripe // ZROWS):
    r0 = s * stripe + j * ZROWS
    pltpu.sync_copy(shared.at[pl.ds(r0, ZROWS)], out.at[c, pl.ds(r0, ZROWS)])


def _combine(y_sorted, tok_padded):
  k = pl.kernel(
      _combine_body,
      out_type=jax.ShapeDtypeStruct((NC, NPAD2, D), jnp.float32),
      mesh=_mesh(),
      scratch_types=[
          pltpu.VMEM((GB,), jnp.int32),
          pltpu.VMEM((GB, D), jnp.float32),
          pltpu.VMEM((ZROWS, D), jnp.float32),
          pltpu.MemorySpace.VMEM_SHARED((NPAD2, D), jnp.float32),
          pltpu.SemaphoreType.DMA,
      ],
  )
  return k(y_sorted, tok_padded)


def _bn_body(h_ref, moe_ref, gamma_ref, beta_ref, out_ref):
  z = h_ref[...] + moe_ref[0, 0:N] + moe_ref[1, 0:N]
  mean = jnp.mean(z, axis=0, keepdims=True)
  zc = z - mean
  var = jnp.mean(zc * zc, axis=0, keepdims=True)
  out_ref[...] = zc * lax.rsqrt(var + 1e-5) * gamma_ref[...][None, :] \
      + beta_ref[...][None, :]


def _batchnorm(h, moe_parts, gamma, beta):
  return pl.pallas_call(
      _bn_body,
      out_shape=jax.ShapeDtypeStruct((N, D), jnp.float32),
  )(h, moe_parts, gamma, beta)


def kernel(feats, edge_index, W_conv, b_conv, gate_W, gate_b, W1, b1, W2, b2,
           bn_gamma, bn_beta):
  src = edge_index[0]
  dst = edge_index[1]
  cnts = _degrees(src, dst)
  hpre, isr_in = _scale(cnts, feats)
  parts = _aggregate(hpre, src, dst)
  h, idx2, gates = _conv_gate(parts, isr_in, W_conv, b_conv, gate_W, gate_b)

  ex = idx2.reshape(A2)
  gflat = gates.reshape(A2)
  perm = jnp.argsort(ex, stable=True).astype(jnp.int32)
  ex_sorted = jnp.take(ex, perm)
  offsets = jnp.searchsorted(ex_sorted, jnp.arange(NUM_EXPERTS, dtype=jnp.int32),
                             side="left").astype(jnp.int32)
  counts = jnp.diff(jnp.append(offsets, A2)).astype(jnp.int32)
  pcounts = ((counts + BB - 1) // BB) * BB
  poffsets = jnp.concatenate([jnp.zeros((1,), jnp.int32),
                              jnp.cumsum(pcounts)]).astype(jnp.int32)
  p = jnp.arange(PB, dtype=jnp.int32)
  e_of_p = jnp.minimum(
      jnp.sum((p[:, None] >= poffsets[None, 1:]).astype(jnp.int32), axis=1),
      NUM_EXPERTS - 1)
  r = p - jnp.take(poffsets, e_of_p)
  valid = r < jnp.take(counts, e_of_p)
  j = jnp.minimum(jnp.take(offsets, e_of_p) + r, A2 - 1)
  a = jnp.take(perm, j)
  tok_padded = jnp.where(valid, a // TOP_K, p % N)
  gate_padded = jnp.where(valid, jnp.take(gflat, a), 0.0)
  block_expert = e_of_p[::BB]

  x_sorted = _gather_rows(h, tok_padded)
  y_sorted = _ffn(x_sorted, gate_padded, block_expert, W1, b1, W2, b2)
  moe_parts = _combine(y_sorted, tok_padded)
  return _batchnorm(h, moe_parts, bn_gamma, bn_beta)


# speedup vs baseline: 1.2941x; 1.2941x over previous
import functools

import jax
import jax.numpy as jnp
from jax import lax
from jax.experimental import pallas as pl
from jax.experimental.pallas import tpu as pltpu
from jax.experimental.pallas import tpu_sc as plsc

N = 10000
E = 320000
D = 128
H = 256
NUM_EXPERTS = 64
TOP_K = 2

NC = 2
NS = 16
LANES = 16
NW = NC * NS

EDGES_PER_TILE = E // NW
EDGE_BATCH = 80
NPAD = 10240
ROWS_PER_TILE = NPAD // NS
ZROWS = 128


def _mesh():
  return plsc.VectorSubcoreMesh(
      core_axis_name="c", subcore_axis_name="s", num_cores=NC,
      num_subcores=NS)


def _degrees_body(src_hbm, dst_hbm, out, sidx, didx, ones_v, zbuf,
                  cnt_out_sh, cnt_in_sh):
  c = lax.axis_index("c")
  s = lax.axis_index("s")

  zeros16 = jnp.zeros((LANES,), jnp.float32)

  def zero_body(i, _):
    zbuf[pl.ds(i * LANES, LANES)] = zeros16
    return 0

  lax.fori_loop(0, N // LANES, zero_body, 0)

  @pl.when(s == 0)
  def _():
    pltpu.sync_copy(zbuf, cnt_out_sh)

  @pl.when(s == 1)
  def _():
    pltpu.sync_copy(zbuf, cnt_in_sh)

  def ones_body(i, _):
    ones_v[pl.ds(i * LANES, LANES)] = jnp.ones((LANES,), jnp.float32)
    return 0

  lax.fori_loop(0, EDGE_BATCH // LANES, ones_body, 0)
  plsc.subcore_barrier()

  def count_body(i, _):
    base = c * (E // NC) + s * EDGES_PER_TILE + i * EDGE_BATCH
    pltpu.sync_copy(src_hbm.at[pl.ds(base, EDGE_BATCH)], sidx)
    pltpu.sync_copy(dst_hbm.at[pl.ds(base, EDGE_BATCH)], didx)
    pltpu.sync_copy(ones_v, cnt_out_sh.at[sidx], add=True)
    pltpu.sync_copy(ones_v, cnt_in_sh.at[didx], add=True)
    return 0

  lax.fori_loop(0, EDGES_PER_TILE // EDGE_BATCH, count_body, 0)
  plsc.subcore_barrier()

  @pl.when(s == 0)
  def _():
    pltpu.sync_copy(cnt_out_sh, out.at[c, 0])

  @pl.when(s == 1)
  def _():
    pltpu.sync_copy(cnt_in_sh, out.at[c, 1])


def _degrees(src, dst):
  k = pl.kernel(
      _degrees_body,
      out_type=jax.ShapeDtypeStruct((NC, 2, N), jnp.float32),
      mesh=_mesh(),
      scratch_types=[
          pltpu.VMEM((EDGE_BATCH,), jnp.int32),
          pltpu.VMEM((EDGE_BATCH,), jnp.int32),
          pltpu.VMEM((EDGE_BATCH,), jnp.float32),
          pltpu.VMEM((N,), jnp.float32),
          pltpu.MemorySpace.VMEM_SHARED((N,), jnp.float32),
          pltpu.MemorySpace.VMEM_SHARED((N,), jnp.float32),
      ],
  )
  return k(src, dst)


def _scale_body(cnts_ref, feats_ref, hpre_ref, isr_in_ref):
  cnts = jnp.sum(cnts_ref[...], axis=0)
  deg = jnp.maximum(cnts, 1.0)
  isr = lax.rsqrt(deg)
  hpre_ref[...] = feats_ref[...] * isr[0][:, None]
  isr_in_ref[...] = isr[1][:, None]


def _scale(cnts, feats):
  return pl.pallas_call(
      _scale_body,
      out_shape=(
          jax.ShapeDtypeStruct((N, D), jnp.float32),
          jax.ShapeDtypeStruct((N, 1), jnp.float32),
      ),
  )(cnts, feats)


def _agg_body(hpre, src_hbm, dst_hbm, out, sidx, didx, rows, zbuf, shared,
              sem):
  c = lax.axis_index("c")
  s = lax.axis_index("s")

  zeros16 = jnp.zeros((LANES,), jnp.float32)

  def zero_body(i, _):
    def zcol(j, _):
      zbuf[i, pl.ds(j * LANES, LANES)] = zeros16
      return 0
    lax.fori_loop(0, D // LANES, zcol, 0)
    return 0

  lax.fori_loop(0, ZROWS, zero_body, 0)
  for j in range(ROWS_PER_TILE // ZROWS):
    pltpu.sync_copy(zbuf, shared.at[pl.ds(s * ROWS_PER_TILE + j * ZROWS, ZROWS)])
  plsc.subcore_barrier()

  def edge_body(i, _):
    base = c * (E // NC) + s * EDGES_PER_TILE + i * EDGE_BATCH
    pltpu.sync_copy(src_hbm.at[pl.ds(base, EDGE_BATCH)], sidx)
    pltpu.sync_copy(dst_hbm.at[pl.ds(base, EDGE_BATCH)], didx)
    pltpu.async_copy(hpre.at[sidx], rows, sem).wait()
    pltpu.sync_copy(rows, shared.at[didx], add=True)
    return 0

  lax.fori_loop(0, EDGES_PER_TILE // EDGE_BATCH, edge_body, 0)
  plsc.subcore_barrier()

  for j in range(ROWS_PER_TILE // ZROWS):
    r0 = s * ROWS_PER_TILE + j * ZROWS
    pltpu.sync_copy(shared.at[pl.ds(r0, ZROWS)], out.at[c, pl.ds(r0, ZROWS)])


def _aggregate(hpre, src, dst):
  k = pl.kernel(
      _agg_body,
      out_type=jax.ShapeDtypeStruct((NC, NPAD, D), jnp.float32),
      mesh=_mesh(),
      scratch_types=[
          pltpu.VMEM((EDGE_BATCH,), jnp.int32),
          pltpu.VMEM((EDGE_BATCH,), jnp.int32),
          pltpu.VMEM((EDGE_BATCH, D), jnp.float32),
          pltpu.VMEM((ZROWS, D), jnp.float32),
          pltpu.MemorySpace.VMEM_SHARED((NPAD, D), jnp.float32),
          pltpu.SemaphoreType.DMA,
      ],
  )
  return k(hpre, src, dst)


def _conv_gate_body(p_ref, isr_ref, wc_ref, bc_ref, gw_ref, gb_ref,
                    h_ref, idx_ref, g_ref):
  agg = (p_ref[0, 0:N] + p_ref[1, 0:N]) * isr_ref[...]
  h = jnp.dot(agg, wc_ref[...], preferred_element_type=jnp.float32)
  h = h + bc_ref[...][None, :]
  h_ref[...] = h
  logits = jnp.dot(h, gw_ref[...], preferred_element_type=jnp.float32)
  logits = logits + gb_ref[...][None, :]
  iota = lax.broadcasted_iota(jnp.int32, (N, NUM_EXPERTS), 1)
  m1 = jnp.max(logits, axis=1, keepdims=True)
  i1 = jnp.min(jnp.where(logits == m1, iota, NUM_EXPERTS), axis=1,
               keepdims=True)
  masked = jnp.where(iota == i1, -jnp.inf, logits)
  m2 = jnp.max(masked, axis=1, keepdims=True)
  i2 = jnp.min(jnp.where(masked == m2, iota, NUM_EXPERTS), axis=1,
               keepdims=True)
  e2 = jnp.exp(m2 - m1)
  denom = 1.0 + e2
  g1 = 1.0 / denom
  g2 = e2 / denom
  idx_ref[...] = jnp.concatenate([i1, i2], axis=1)
  g_ref[...] = jnp.concatenate([g1, g2], axis=1)


def _conv_gate(parts, isr_in, W_conv, b_conv, gate_W, gate_b):
  return pl.pallas_call(
      _conv_gate_body,
      out_shape=(
          jax.ShapeDtypeStruct((N, D), jnp.float32),
          jax.ShapeDtypeStruct((N, TOP_K), jnp.int32),
          jax.ShapeDtypeStruct((N, TOP_K), jnp.float32),
      ),
  )(parts, isr_in, W_conv, b_conv, gate_W, gate_b)


A2 = N * TOP_K
BB = 256
PB = 36864
NB = PB // BB
GB = 128
GROWS = PB // NW
NPAD2 = 10240
CTOK = NPAD2 // NW
CB = 40


def _gather_rows_body(h_hbm, tok_hbm, out, tidx, rows, sem):
  c = lax.axis_index("c")
  s = lax.axis_index("s")
  w = s * NC + c
  base = w * GROWS

  def body(i, _):
    r0 = base + i * GB
    pltpu.sync_copy(tok_hbm.at[pl.ds(r0, GB)], tidx)
    pltpu.async_copy(h_hbm.at[tidx], rows, sem).wait()
    pltpu.sync_copy(rows, out.at[pl.ds(r0, GB)])
    return 0

  lax.fori_loop(0, GROWS // GB, body, 0)


def _gather_rows(h, tok_padded):
  k = pl.kernel(
      _gather_rows_body,
      out_type=jax.ShapeDtypeStruct((PB, D), jnp.float32),
      mesh=_mesh(),
      scratch_types=[
          pltpu.VMEM((GB,), jnp.int32),
          pltpu.VMEM((GB, D), jnp.float32),
          pltpu.SemaphoreType.DMA,
      ],
  )
  return k(h, tok_padded)


def _ffn_body(be_ref, x_ref, g_ref, w1_ref, b1_ref, w2_ref, b2_ref, y_ref):
  x = x_ref[...]
  a = jnp.dot(x, w1_ref[0], preferred_element_type=jnp.float32)
  a = a + b1_ref[0]
  a = 0.5 * a * (1.0 + lax.erf(a * 0.7071067811865476))
  y = jnp.dot(a, w2_ref[0], preferred_element_type=jnp.float32)
  y = y + b2_ref[0]
  y_ref[...] = y * g_ref[...]


def _ffn(x_sorted, gate_padded, block_expert, W1, b1, W2, b2):
  grid_spec = pltpu.PrefetchScalarGridSpec(
      num_scalar_prefetch=1,
      grid=(NB,),
      in_specs=[
          pl.BlockSpec((BB, D), lambda b, be: (b, 0)),
          pl.BlockSpec((BB, 1), lambda b, be: (b, 0)),
          pl.BlockSpec((1, D, H), lambda b, be: (be[b], 0, 0)),
          pl.BlockSpec((1, 1, H), lambda b, be: (be[b], 0, 0)),
          pl.BlockSpec((1, H, D), lambda b, be: (be[b], 0, 0)),
          pl.BlockSpec((1, 1, D), lambda b, be: (be[b], 0, 0)),
      ],
      out_specs=pl.BlockSpec((BB, D), lambda b, be: (b, 0)),
  )
  return pl.pallas_call(
      _ffn_body,
      grid_spec=grid_spec,
      out_shape=jax.ShapeDtypeStruct((PB, D), jnp.float32),
  )(block_expert, x_sorted, gate_padded.reshape(PB, 1),
    W1, b1.reshape(NUM_EXPERTS, 1, H), W2, b2.reshape(NUM_EXPERTS, 1, D))


def _combine_body(y_hbm, tok_hbm, out, tidx, rows, zbuf, shared, sem):
  c = lax.axis_index("c")
  s = lax.axis_index("s")
  w = s * NC + c

  zeros16 = jnp.zeros((LANES,), jnp.float32)

  def zero_body(i, _):
    def zcol(j, _):
      zbuf[i, pl.ds(j * LANES, LANES)] = zeros16
      return 0
    lax.fori_loop(0, D // LANES, zcol, 0)
    return 0

  lax.fori_loop(0, ZROWS, zero_body, 0)
  stripe = NPAD2 // NS
  for j in range(stripe // ZROWS):
    pltpu.sync_copy(zbuf, shared.at[pl.ds(s * stripe + j * ZROWS, ZROWS)])
  plsc.subcore_barrier()

  def body(i, _):
    r0 = w * GROWS + i * GB
    pltpu.sync_copy(y_hbm.at[pl.ds(r0, GB)], rows)
    pltpu.sync_copy(tok_hbm.at[pl.ds(r0, GB)], tidx)
    pltpu.sync_copy(rows, shared.at[tidx], add=True)
    return 0

  lax.fori_loop(0, GROWS // GB, body, 0)
  plsc.subcore_barrier()

  for j in range(stripe // ZROWS):
    r0 = s * stripe + j * ZROWS
    pltpu.sync_copy(shared.at[pl.ds(r0, ZROWS)], out.at[c, pl.ds(r0, ZROWS)])


def _combine(y_sorted, tok_padded):
  k = pl.kernel(
      _combine_body,
      out_type=jax.ShapeDtypeStruct((NC, NPAD2, D), jnp.float32),
      mesh=_mesh(),
      scratch_types=[
          pltpu.VMEM((GB,), jnp.int32),
          pltpu.VMEM((GB, D), jnp.float32),
          pltpu.VMEM((ZROWS, D), jnp.float32),
          pltpu.MemorySpace.VMEM_SHARED((NPAD2, D), jnp.float32),
          pltpu.SemaphoreType.DMA,
      ],
  )
  return k(y_sorted, tok_padded)


def _bn_body(h_ref, moe_ref, gamma_ref, beta_ref, out_ref):
  z = h_ref[...] + moe_ref[0, 0:N] + moe_ref[1, 0:N]
  mean = jnp.mean(z, axis=0, keepdims=True)
  zc = z - mean
  var = jnp.mean(zc * zc, axis=0, keepdims=True)
  out_ref[...] = zc * lax.rsqrt(var + 1e-5) * gamma_ref[...][None, :] \
      + beta_ref[...][None, :]


def _batchnorm(h, moe_parts, gamma, beta):
  return pl.pallas_call(
      _bn_body,
      out_shape=jax.ShapeDtypeStruct((N, D), jnp.float32),
  )(h, moe_parts, gamma, beta)


def kernel(feats, edge_index, W_conv, b_conv, gate_W, gate_b, W1, b1, W2, b2,
           bn_gamma, bn_beta):
  src = edge_index[0]
  dst = edge_index[1]
  cnts = _degrees(src, dst)
  hpre, isr_in = _scale(cnts, feats)
  parts = _aggregate(hpre, src, dst)
  h, idx2, gates = _conv_gate(parts, isr_in, W_conv, b_conv, gate_W, gate_b)

  ex = idx2.reshape(A2)
  gflat = gates.reshape(A2)
  perm = jnp.argsort(ex, stable=True).astype(jnp.int32)
  ex_sorted = jnp.take(ex, perm)
  offsets = jnp.searchsorted(ex_sorted, jnp.arange(NUM_EXPERTS, dtype=jnp.int32),
                             side="left").astype(jnp.int32)
  counts = jnp.diff(jnp.append(offsets, A2)).astype(jnp.int32)
  pcounts = ((counts + BB - 1) // BB) * BB
  poffsets = jnp.concatenate([jnp.zeros((1,), jnp.int32),
                              jnp.cumsum(pcounts)]).astype(jnp.int32)
  block_expert = jnp.clip(
      jnp.searchsorted(poffsets, jnp.arange(NB, dtype=jnp.int32) * BB,
                       side="right").astype(jnp.int32) - 1, 0, NUM_EXPERTS - 1)
  p = jnp.arange(PB, dtype=jnp.int32)
  r = p - jnp.repeat(jnp.take(poffsets, block_expert), BB)
  valid = r < jnp.repeat(jnp.take(counts, block_expert), BB)
  j = jnp.minimum(jnp.repeat(jnp.take(offsets, block_expert), BB) + r, A2 - 1)
  a = jnp.take(perm, j)
  tok_padded = jnp.where(valid, a // TOP_K, p & 8191)
  gate_padded = jnp.where(valid, jnp.take(gflat, a), 0.0)

  x_sorted = _gather_rows(h, tok_padded)
  y_sorted = _ffn(x_sorted, gate_padded, block_expert, W1, b1, W2, b2)
  moe_parts = _combine(y_sorted, tok_padded)
  return _batchnorm(h, moe_parts, bn_gamma, bn_beta)


# speedup vs baseline: 1.6665x; 1.2877x over previous
import functools

import jax
import jax.numpy as jnp
from jax import lax
from jax.experimental import pallas as pl
from jax.experimental.pallas import tpu as pltpu
from jax.experimental.pallas import tpu_sc as plsc

N = 10000
E = 320000
D = 128
H = 256
NUM_EXPERTS = 64
TOP_K = 2

NC = 2
NS = 16
LANES = 16
NW = NC * NS

EDGES_PER_TILE = E // NW
EDGE_BATCH = 128
NEB = E // EDGE_BATCH
NEB_MAIN = NEB // NW
NPAD = 10240
ROWS_PER_TILE = NPAD // NS
ZROWS = 64


def _mesh():
  return plsc.VectorSubcoreMesh(
      core_axis_name="c", subcore_axis_name="s", num_cores=NC,
      num_subcores=NS)


def _degrees_body(eflat, out, sidx, didx, ones_v, zbuf,
                  cnt_out_sh, cnt_in_sh):
  c = lax.axis_index("c")
  s = lax.axis_index("s")
  w = s * NC + c

  zeros16 = jnp.zeros((LANES,), jnp.float32)

  def zero_body(i, _):
    zbuf[pl.ds(i * LANES, LANES)] = zeros16
    return 0

  lax.fori_loop(0, N // LANES, zero_body, 0)

  @pl.when(s == 0)
  def _():
    pltpu.sync_copy(zbuf, cnt_out_sh)

  @pl.when(s == 1)
  def _():
    pltpu.sync_copy(zbuf, cnt_in_sh)

  def ones_body(i, _):
    ones_v[pl.ds(i * LANES, LANES)] = jnp.ones((LANES,), jnp.float32)
    return 0

  lax.fori_loop(0, EDGE_BATCH // LANES, ones_body, 0)
  plsc.subcore_barrier()

  def count_batch(b):
    base = b * EDGE_BATCH
    pltpu.sync_copy(eflat.at[pl.ds(base, EDGE_BATCH)], sidx)
    pltpu.sync_copy(eflat.at[pl.ds(E + base, EDGE_BATCH)], didx)
    pltpu.sync_copy(ones_v, cnt_out_sh.at[sidx], add=True)
    pltpu.sync_copy(ones_v, cnt_in_sh.at[didx], add=True)

  def count_body(i, _):
    count_batch(w * NEB_MAIN + i)
    return 0

  lax.fori_loop(0, NEB_MAIN, count_body, 0)

  @pl.when(w < NEB - NW * NEB_MAIN)
  def _():
    count_batch(NW * NEB_MAIN + w)

  plsc.subcore_barrier()

  @pl.when(s == 0)
  def _():
    pltpu.sync_copy(cnt_out_sh, out.at[c, 0])

  @pl.when(s == 1)
  def _():
    pltpu.sync_copy(cnt_in_sh, out.at[c, 1])


def _degrees(eflat):
  k = pl.kernel(
      _degrees_body,
      out_type=jax.ShapeDtypeStruct((NC, 2, N), jnp.float32),
      mesh=_mesh(),
      scratch_types=[
          pltpu.VMEM((EDGE_BATCH,), jnp.int32),
          pltpu.VMEM((EDGE_BATCH,), jnp.int32),
          pltpu.VMEM((EDGE_BATCH,), jnp.float32),
          pltpu.VMEM((N,), jnp.float32),
          pltpu.MemorySpace.VMEM_SHARED((N,), jnp.float32),
          pltpu.MemorySpace.VMEM_SHARED((N,), jnp.float32),
      ],
  )
  return k(eflat)


def _scale_body(cnts_ref, feats_ref, hpre_ref, isr_in_ref):
  cnts = jnp.sum(cnts_ref[...], axis=0)
  deg = jnp.maximum(cnts, 1.0)
  isr = lax.rsqrt(deg)
  hpre_ref[...] = feats_ref[...] * isr[0][:, None]
  isr_in_ref[...] = isr[1][:, None]


def _scale(cnts, feats):
  return pl.pallas_call(
      _scale_body,
      out_shape=(
          jax.ShapeDtypeStruct((N, D), jnp.float32),
          jax.ShapeDtypeStruct((N, 1), jnp.float32),
      ),
  )(cnts, feats)


def _agg_body(hpre, eflat, out, sidx_a, didx_a, rows_a, sidx_b, didx_b,
              rows_b, zbuf, shared, sem_a, sem_b):
  c = lax.axis_index("c")
  s = lax.axis_index("s")
  w = s * NC + c

  zeros16 = jnp.zeros((LANES,), jnp.float32)

  def zero_body(i, _):
    def zcol(j, _):
      zbuf[i, pl.ds(j * LANES, LANES)] = zeros16
      return 0
    lax.fori_loop(0, D // LANES, zcol, 0)
    return 0

  lax.fori_loop(0, ZROWS, zero_body, 0)
  for j in range(ROWS_PER_TILE // ZROWS):
    pltpu.sync_copy(zbuf, shared.at[pl.ds(s * ROWS_PER_TILE + j * ZROWS, ZROWS)])
  plsc.subcore_barrier()

  def fetch(b, sidx, didx, rows, sem):
    base = b * EDGE_BATCH
    pltpu.sync_copy(eflat.at[pl.ds(base, EDGE_BATCH)], sidx)
    pltpu.sync_copy(eflat.at[pl.ds(E + base, EDGE_BATCH)], didx)
    pltpu.async_copy(hpre.at[sidx], rows, sem)

  def drain_scatter(sidx, didx, rows, sem):
    pltpu.make_async_copy(hpre.at[sidx], rows, sem).wait()
    pltpu.sync_copy(rows, shared.at[didx], add=True)

  b0 = w * NEB_MAIN
  fetch(b0, sidx_a, didx_a, rows_a, sem_a)

  def edge_body(k, _):
    b = b0 + 2 * k
    fetch(b + 1, sidx_b, didx_b, rows_b, sem_b)
    drain_scatter(sidx_a, didx_a, rows_a, sem_a)

    @pl.when(k < NEB_MAIN // 2 - 1)
    def _():
      fetch(b + 2, sidx_a, didx_a, rows_a, sem_a)

    drain_scatter(sidx_b, didx_b, rows_b, sem_b)
    return 0

  lax.fori_loop(0, NEB_MAIN // 2, edge_body, 0)

  @pl.when(w < NEB - NW * NEB_MAIN)
  def _():
    fetch(NW * NEB_MAIN + w, sidx_a, didx_a, rows_a, sem_a)
    drain_scatter(sidx_a, didx_a, rows_a, sem_a)

  plsc.subcore_barrier()

  for j in range(ROWS_PER_TILE // ZROWS):
    r0 = s * ROWS_PER_TILE + j * ZROWS
    pltpu.sync_copy(shared.at[pl.ds(r0, ZROWS)], out.at[c, pl.ds(r0, ZROWS)])


def _aggregate(hpre, eflat):
  k = pl.kernel(
      _agg_body,
      out_type=jax.ShapeDtypeStruct((NC, NPAD, D), jnp.float32),
      mesh=_mesh(),
      scratch_types=[
          pltpu.VMEM((EDGE_BATCH,), jnp.int32),
          pltpu.VMEM((EDGE_BATCH,), jnp.int32),
          pltpu.VMEM((EDGE_BATCH, D), jnp.float32),
          pltpu.VMEM((EDGE_BATCH,), jnp.int32),
          pltpu.VMEM((EDGE_BATCH,), jnp.int32),
          pltpu.VMEM((EDGE_BATCH, D), jnp.float32),
          pltpu.VMEM((ZROWS, D), jnp.float32),
          pltpu.MemorySpace.VMEM_SHARED((NPAD, D), jnp.float32),
          pltpu.SemaphoreType.DMA,
          pltpu.SemaphoreType.DMA,
      ],
  )
  return k(hpre, eflat)


def _conv_gate_body(p_ref, isr_ref, wc_ref, bc_ref, gw_ref, gb_ref,
                    h_ref, idx_ref, g_ref):
  agg = (p_ref[0, 0:N] + p_ref[1, 0:N]) * isr_ref[...]
  h = jnp.dot(agg, wc_ref[...], preferred_element_type=jnp.float32)
  h = h + bc_ref[...][None, :]
  h_ref[...] = h
  logits = jnp.dot(h, gw_ref[...], preferred_element_type=jnp.float32)
  logits = logits + gb_ref[...][None, :]
  iota = lax.broadcasted_iota(jnp.int32, (N, NUM_EXPERTS), 1)
  m1 = jnp.max(logits, axis=1, keepdims=True)
  i1 = jnp.min(jnp.where(logits == m1, iota, NUM_EXPERTS), axis=1,
               keepdims=True)
  masked = jnp.where(iota == i1, -jnp.inf, logits)
  m2 = jnp.max(masked, axis=1, keepdims=True)
  i2 = jnp.min(jnp.where(masked == m2, iota, NUM_EXPERTS), axis=1,
               keepdims=True)
  e2 = jnp.exp(m2 - m1)
  denom = 1.0 + e2
  g1 = 1.0 / denom
  g2 = e2 / denom
  idx_ref[...] = jnp.concatenate([i1, i2], axis=1)
  g_ref[...] = jnp.concatenate([g1, g2], axis=1)


def _conv_gate(parts, isr_in, W_conv, b_conv, gate_W, gate_b):
  return pl.pallas_call(
      _conv_gate_body,
      out_shape=(
          jax.ShapeDtypeStruct((N, D), jnp.float32),
          jax.ShapeDtypeStruct((N, TOP_K), jnp.int32),
          jax.ShapeDtypeStruct((N, TOP_K), jnp.float32),
      ),
  )(parts, isr_in, W_conv, b_conv, gate_W, gate_b)


A2 = N * TOP_K
BB = 256
PB = 36864
NB = PB // BB
GB = 128
GROWS = PB // NW
NPAD2 = 10240
CTOK = NPAD2 // NW
CB = 40


def _gather_rows_body(h_hbm, tok_hbm, out, tidx, rows, sem):
  c = lax.axis_index("c")
  s = lax.axis_index("s")
  w = s * NC + c
  base = w * GROWS

  def body(i, _):
    r0 = base + i * GB
    pltpu.sync_copy(tok_hbm.at[pl.ds(r0, GB)], tidx)
    pltpu.async_copy(h_hbm.at[tidx], rows, sem).wait()
    pltpu.sync_copy(rows, out.at[pl.ds(r0, GB)])
    return 0

  lax.fori_loop(0, GROWS // GB, body, 0)


def _gather_rows(h, tok_padded):
  k = pl.kernel(
      _gather_rows_body,
      out_type=jax.ShapeDtypeStruct((PB, D), jnp.float32),
      mesh=_mesh(),
      scratch_types=[
          pltpu.VMEM((GB,), jnp.int32),
          pltpu.VMEM((GB, D), jnp.float32),
          pltpu.SemaphoreType.DMA,
      ],
  )
  return k(h, tok_padded)


def _ffn_body(be_ref, x_ref, g_ref, w1_ref, b1_ref, w2_ref, b2_ref, y_ref):
  x = x_ref[...]
  a = jnp.dot(x, w1_ref[0], preferred_element_type=jnp.float32)
  a = a + b1_ref[0]
  a = 0.5 * a * (1.0 + lax.erf(a * 0.7071067811865476))
  y = jnp.dot(a, w2_ref[0], preferred_element_type=jnp.float32)
  y = y + b2_ref[0]
  y_ref[...] = y * g_ref[...]


def _ffn(x_sorted, gate_padded, block_expert, W1, b1, W2, b2):
  grid_spec = pltpu.PrefetchScalarGridSpec(
      num_scalar_prefetch=1,
      grid=(NB,),
      in_specs=[
          pl.BlockSpec((BB, D), lambda b, be: (b, 0)),
          pl.BlockSpec((BB, 1), lambda b, be: (b, 0)),
          pl.BlockSpec((1, D, H), lambda b, be: (be[b], 0, 0)),
          pl.BlockSpec((1, 1, H), lambda b, be: (be[b], 0, 0)),
          pl.BlockSpec((1, H, D), lambda b, be: (be[b], 0, 0)),
          pl.BlockSpec((1, 1, D), lambda b, be: (be[b], 0, 0)),
      ],
      out_specs=pl.BlockSpec((BB, D), lambda b, be: (b, 0)),
  )
  return pl.pallas_call(
      _ffn_body,
      grid_spec=grid_spec,
      out_shape=jax.ShapeDtypeStruct((PB, D), jnp.float32),
  )(block_expert, x_sorted, gate_padded.reshape(PB, 1),
    W1, b1.reshape(NUM_EXPERTS, 1, H), W2, b2.reshape(NUM_EXPERTS, 1, D))


def _combine_body(y_hbm, tok_hbm, out, tidx, rows, zbuf, shared, sem):
  c = lax.axis_index("c")
  s = lax.axis_index("s")
  w = s * NC + c

  zeros16 = jnp.zeros((LANES,), jnp.float32)

  def zero_body(i, _):
    def zcol(j, _):
      zbuf[i, pl.ds(j * LANES, LANES)] = zeros16
      return 0
    lax.fori_loop(0, D // LANES, zcol, 0)
    return 0

  lax.fori_loop(0, ZROWS, zero_body, 0)
  stripe = NPAD2 // NS
  for j in range(stripe // ZROWS):
    pltpu.sync_copy(zbuf, shared.at[pl.ds(s * stripe + j * ZROWS, ZROWS)])
  plsc.subcore_barrier()

  def body(i, _):
    r0 = w * GROWS + i * GB
    pltpu.sync_copy(y_hbm.at[pl.ds(r0, GB)], rows)
    pltpu.sync_copy(tok_hbm.at[pl.ds(r0, GB)], tidx)
    pltpu.sync_copy(rows, shared.at[tidx], add=True)
    return 0

  lax.fori_loop(0, GROWS // GB, body, 0)
  plsc.subcore_barrier()

  for j in range(stripe // ZROWS):
    r0 = s * stripe + j * ZROWS
    pltpu.sync_copy(shared.at[pl.ds(r0, ZROWS)], out.at[c, pl.ds(r0, ZROWS)])


def _combine(y_sorted, tok_padded):
  k = pl.kernel(
      _combine_body,
      out_type=jax.ShapeDtypeStruct((NC, NPAD2, D), jnp.float32),
      mesh=_mesh(),
      scratch_types=[
          pltpu.VMEM((GB,), jnp.int32),
          pltpu.VMEM((GB, D), jnp.float32),
          pltpu.VMEM((ZROWS, D), jnp.float32),
          pltpu.MemorySpace.VMEM_SHARED((NPAD2, D), jnp.float32),
          pltpu.SemaphoreType.DMA,
      ],
  )
  return k(y_sorted, tok_padded)


def _bn_body(h_ref, moe_ref, gamma_ref, beta_ref, out_ref):
  z = h_ref[...] + moe_ref[0, 0:N] + moe_ref[1, 0:N]
  mean = jnp.mean(z, axis=0, keepdims=True)
  zc = z - mean
  var = jnp.mean(zc * zc, axis=0, keepdims=True)
  out_ref[...] = zc * lax.rsqrt(var + 1e-5) * gamma_ref[...][None, :] \
      + beta_ref[...][None, :]


def _batchnorm(h, moe_parts, gamma, beta):
  return pl.pallas_call(
      _bn_body,
      out_shape=jax.ShapeDtypeStruct((N, D), jnp.float32),
  )(h, moe_parts, gamma, beta)


def kernel(feats, edge_index, W_conv, b_conv, gate_W, gate_b, W1, b1, W2, b2,
           bn_gamma, bn_beta):
  eflat = edge_index.reshape(2 * E)
  cnts = _degrees(eflat)
  hpre, isr_in = _scale(cnts, feats)
  parts = _aggregate(hpre, eflat)
  h, idx2, gates = _conv_gate(parts, isr_in, W_conv, b_conv, gate_W, gate_b)

  ex = idx2.reshape(A2)
  gflat = gates.reshape(A2)
  perm = jnp.argsort(ex, stable=True).astype(jnp.int32)
  ex_sorted = jnp.take(ex, perm)
  offsets = jnp.searchsorted(ex_sorted, jnp.arange(NUM_EXPERTS, dtype=jnp.int32),
                             side="left").astype(jnp.int32)
  counts = jnp.diff(jnp.append(offsets, A2)).astype(jnp.int32)
  pcounts = ((counts + BB - 1) // BB) * BB
  poffsets = jnp.concatenate([jnp.zeros((1,), jnp.int32),
                              jnp.cumsum(pcounts)]).astype(jnp.int32)
  block_expert = jnp.clip(
      jnp.searchsorted(poffsets, jnp.arange(NB, dtype=jnp.int32) * BB,
                       side="right").astype(jnp.int32) - 1, 0, NUM_EXPERTS - 1)
  p = jnp.arange(PB, dtype=jnp.int32)
  r = p - jnp.repeat(jnp.take(poffsets, block_expert), BB)
  valid = r < jnp.repeat(jnp.take(counts, block_expert), BB)
  j = jnp.minimum(jnp.repeat(jnp.take(offsets, block_expert), BB) + r, A2 - 1)
  a = jnp.take(perm, j)
  tok_padded = jnp.where(valid, a // TOP_K, p & 8191)
  gate_padded = jnp.where(valid, jnp.take(gflat, a), 0.0)

  x_sorted = _gather_rows(h, tok_padded)
  y_sorted = _ffn(x_sorted, gate_padded, block_expert, W1, b1, W2, b2)
  moe_parts = _combine(y_sorted, tok_padded)
  return _batchnorm(h, moe_parts, bn_gamma, bn_beta)


# speedup vs baseline: 1.7172x; 1.0304x over previous
import functools

import jax
import jax.numpy as jnp
from jax import lax
from jax.experimental import pallas as pl
from jax.experimental.pallas import tpu as pltpu
from jax.experimental.pallas import tpu_sc as plsc

N = 10000
E = 320000
D = 128
H = 256
NUM_EXPERTS = 64
TOP_K = 2

NC = 2
NS = 16
LANES = 16
NW = NC * NS

EDGES_PER_TILE = E // NW
EDGE_BATCH = 128
NEB = E // EDGE_BATCH
NEB_MAIN = NEB // NW
NPAD = 10240
ROWS_PER_TILE = NPAD // NS
ZROWS = 64


def _mesh():
  return plsc.VectorSubcoreMesh(
      core_axis_name="c", subcore_axis_name="s", num_cores=NC,
      num_subcores=NS)


def _degrees_body(eflat, out, sidx, didx, ones_v, zbuf,
                  cnt_out_sh, cnt_in_sh):
  c = lax.axis_index("c")
  s = lax.axis_index("s")
  w = s * NC + c

  zeros16 = jnp.zeros((LANES,), jnp.float32)

  def zero_body(i, _):
    zbuf[pl.ds(i * LANES, LANES)] = zeros16
    return 0

  lax.fori_loop(0, N // LANES, zero_body, 0)

  @pl.when(s == 0)
  def _():
    pltpu.sync_copy(zbuf, cnt_out_sh)

  @pl.when(s == 1)
  def _():
    pltpu.sync_copy(zbuf, cnt_in_sh)

  def ones_body(i, _):
    ones_v[pl.ds(i * LANES, LANES)] = jnp.ones((LANES,), jnp.float32)
    return 0

  lax.fori_loop(0, EDGE_BATCH // LANES, ones_body, 0)
  plsc.subcore_barrier()

  def count_batch(b):
    base = b * EDGE_BATCH
    pltpu.sync_copy(eflat.at[pl.ds(base, EDGE_BATCH)], sidx)
    pltpu.sync_copy(eflat.at[pl.ds(E + base, EDGE_BATCH)], didx)
    pltpu.sync_copy(ones_v, cnt_out_sh.at[sidx], add=True)
    pltpu.sync_copy(ones_v, cnt_in_sh.at[didx], add=True)

  def count_body(i, _):
    count_batch(w * NEB_MAIN + i)
    return 0

  lax.fori_loop(0, NEB_MAIN, count_body, 0)

  @pl.when(w < NEB - NW * NEB_MAIN)
  def _():
    count_batch(NW * NEB_MAIN + w)

  plsc.subcore_barrier()

  @pl.when(s == 0)
  def _():
    pltpu.sync_copy(cnt_out_sh, out.at[c, 0])

  @pl.when(s == 1)
  def _():
    pltpu.sync_copy(cnt_in_sh, out.at[c, 1])


def _degrees(eflat):
  k = pl.kernel(
      _degrees_body,
      out_type=jax.ShapeDtypeStruct((NC, 2, N), jnp.float32),
      mesh=_mesh(),
      scratch_types=[
          pltpu.VMEM((EDGE_BATCH,), jnp.int32),
          pltpu.VMEM((EDGE_BATCH,), jnp.int32),
          pltpu.VMEM((EDGE_BATCH,), jnp.float32),
          pltpu.VMEM((N,), jnp.float32),
          pltpu.MemorySpace.VMEM_SHARED((N,), jnp.float32),
          pltpu.MemorySpace.VMEM_SHARED((N,), jnp.float32),
      ],
  )
  return k(eflat)


def _scale_body(cnts_ref, feats_ref, hpre_ref, isr_in_ref):
  cnts = jnp.sum(cnts_ref[...], axis=0)
  deg = jnp.maximum(cnts, 1.0)
  isr = lax.rsqrt(deg)
  hpre_ref[...] = feats_ref[...] * isr[0][:, None]
  isr_in_ref[...] = isr[1][:, None]


def _scale(cnts, feats):
  return pl.pallas_call(
      _scale_body,
      out_shape=(
          jax.ShapeDtypeStruct((N, D), jnp.float32),
          jax.ShapeDtypeStruct((N, 1), jnp.float32),
      ),
  )(cnts, feats)


def _agg_body(hpre, eflat, out, sidx_a, didx_a, rows_a, sidx_b, didx_b,
              rows_b, zbuf, shared, sem_a, sem_b):
  c = lax.axis_index("c")
  s = lax.axis_index("s")
  w = s * NC + c

  zeros16 = jnp.zeros((LANES,), jnp.float32)

  def zero_body(i, _):
    def zcol(j, _):
      zbuf[i, pl.ds(j * LANES, LANES)] = zeros16
      return 0
    lax.fori_loop(0, D // LANES, zcol, 0)
    return 0

  lax.fori_loop(0, ZROWS, zero_body, 0)
  for j in range(ROWS_PER_TILE // ZROWS):
    pltpu.sync_copy(zbuf, shared.at[pl.ds(s * ROWS_PER_TILE + j * ZROWS, ZROWS)])
  plsc.subcore_barrier()

  def fetch(b, sidx, didx, rows, sem):
    base = b * EDGE_BATCH
    pltpu.sync_copy(eflat.at[pl.ds(base, EDGE_BATCH)], sidx)
    pltpu.sync_copy(eflat.at[pl.ds(E + base, EDGE_BATCH)], didx)
    pltpu.async_copy(hpre.at[sidx], rows, sem)

  def drain_scatter(sidx, didx, rows, sem):
    pltpu.make_async_copy(hpre.at[sidx], rows, sem).wait()
    pltpu.sync_copy(rows, shared.at[didx], add=True)

  b0 = w * NEB_MAIN
  fetch(b0, sidx_a, didx_a, rows_a, sem_a)

  def edge_body(k, _):
    b = b0 + 2 * k
    fetch(b + 1, sidx_b, didx_b, rows_b, sem_b)
    drain_scatter(sidx_a, didx_a, rows_a, sem_a)

    @pl.when(k < NEB_MAIN // 2 - 1)
    def _():
      fetch(b + 2, sidx_a, didx_a, rows_a, sem_a)

    drain_scatter(sidx_b, didx_b, rows_b, sem_b)
    return 0

  lax.fori_loop(0, NEB_MAIN // 2, edge_body, 0)

  @pl.when(w < NEB - NW * NEB_MAIN)
  def _():
    fetch(NW * NEB_MAIN + w, sidx_a, didx_a, rows_a, sem_a)
    drain_scatter(sidx_a, didx_a, rows_a, sem_a)

  plsc.subcore_barrier()

  for j in range(ROWS_PER_TILE // ZROWS):
    r0 = s * ROWS_PER_TILE + j * ZROWS
    pltpu.sync_copy(shared.at[pl.ds(r0, ZROWS)], out.at[c, pl.ds(r0, ZROWS)])


def _aggregate(hpre, eflat):
  k = pl.kernel(
      _agg_body,
      out_type=jax.ShapeDtypeStruct((NC, NPAD, D), jnp.float32),
      mesh=_mesh(),
      scratch_types=[
          pltpu.VMEM((EDGE_BATCH,), jnp.int32),
          pltpu.VMEM((EDGE_BATCH,), jnp.int32),
          pltpu.VMEM((EDGE_BATCH, D), jnp.float32),
          pltpu.VMEM((EDGE_BATCH,), jnp.int32),
          pltpu.VMEM((EDGE_BATCH,), jnp.int32),
          pltpu.VMEM((EDGE_BATCH, D), jnp.float32),
          pltpu.VMEM((ZROWS, D), jnp.float32),
          pltpu.MemorySpace.VMEM_SHARED((NPAD, D), jnp.float32),
          pltpu.SemaphoreType.DMA,
          pltpu.SemaphoreType.DMA,
      ],
  )
  return k(hpre, eflat)


def _conv_gate_body(p_ref, isr_ref, wc_ref, bc_ref, gw_ref, gb_ref,
                    h_ref, idx_ref, g_ref):
  agg = (p_ref[0, 0:N] + p_ref[1, 0:N]) * isr_ref[...]
  h = jnp.dot(agg, wc_ref[...], preferred_element_type=jnp.float32)
  h = h + bc_ref[...][None, :]
  h_ref[...] = h
  logits = jnp.dot(h, gw_ref[...], preferred_element_type=jnp.float32)
  logits = logits + gb_ref[...][None, :]
  iota = lax.broadcasted_iota(jnp.int32, (N, NUM_EXPERTS), 1)
  m1 = jnp.max(logits, axis=1, keepdims=True)
  i1 = jnp.min(jnp.where(logits == m1, iota, NUM_EXPERTS), axis=1,
               keepdims=True)
  masked = jnp.where(iota == i1, -jnp.inf, logits)
  m2 = jnp.max(masked, axis=1, keepdims=True)
  i2 = jnp.min(jnp.where(masked == m2, iota, NUM_EXPERTS), axis=1,
               keepdims=True)
  e2 = jnp.exp(m2 - m1)
  denom = 1.0 + e2
  g1 = 1.0 / denom
  g2 = e2 / denom
  idx_ref[...] = jnp.concatenate([i1, i2], axis=1)
  g_ref[...] = jnp.concatenate([g1, g2], axis=1)


def _conv_gate(parts, isr_in, W_conv, b_conv, gate_W, gate_b):
  return pl.pallas_call(
      _conv_gate_body,
      out_shape=(
          jax.ShapeDtypeStruct((N, D), jnp.float32),
          jax.ShapeDtypeStruct((N, TOP_K), jnp.int32),
          jax.ShapeDtypeStruct((N, TOP_K), jnp.float32),
      ),
  )(parts, isr_in, W_conv, b_conv, gate_W, gate_b)


A2 = N * TOP_K
BB = 128
PB = 28672
NB = PB // BB
GB = 128
GROWS = PB // NW
NPAD2 = 10240
CTOK = NPAD2 // NW
CB = 40


def _gather_rows_body(h_hbm, tok_hbm, out, tidx, rows, sem):
  c = lax.axis_index("c")
  s = lax.axis_index("s")
  w = s * NC + c
  base = w * GROWS

  def body(i, _):
    r0 = base + i * GB
    pltpu.sync_copy(tok_hbm.at[pl.ds(r0, GB)], tidx)
    pltpu.async_copy(h_hbm.at[tidx], rows, sem).wait()
    pltpu.sync_copy(rows, out.at[pl.ds(r0, GB)])
    return 0

  lax.fori_loop(0, GROWS // GB, body, 0)


def _gather_rows(h, tok_padded):
  k = pl.kernel(
      _gather_rows_body,
      out_type=jax.ShapeDtypeStruct((PB, D), jnp.float32),
      mesh=_mesh(),
      scratch_types=[
          pltpu.VMEM((GB,), jnp.int32),
          pltpu.VMEM((GB, D), jnp.float32),
          pltpu.SemaphoreType.DMA,
      ],
  )
  return k(h, tok_padded)


def _ffn_body(be_ref, x_ref, g_ref, w1_ref, b1_ref, w2_ref, b2_ref, y_ref):
  x = x_ref[...]
  a = jnp.dot(x.astype(jnp.bfloat16), w1_ref[0],
              preferred_element_type=jnp.float32)
  a = a + b1_ref[0]
  a = 0.5 * a * (1.0 + lax.erf(a * 0.7071067811865476))
  y = jnp.dot(a.astype(jnp.bfloat16), w2_ref[0],
              preferred_element_type=jnp.float32)
  y = y + b2_ref[0]
  y_ref[...] = y * g_ref[...]


def _ffn(x_sorted, gate_padded, block_expert, W1, b1, W2, b2):
  grid_spec = pltpu.PrefetchScalarGridSpec(
      num_scalar_prefetch=1,
      grid=(NB,),
      in_specs=[
          pl.BlockSpec((BB, D), lambda b, be: (b, 0)),
          pl.BlockSpec((BB, 1), lambda b, be: (b, 0)),
          pl.BlockSpec((1, D, H), lambda b, be: (be[b], 0, 0)),
          pl.BlockSpec((1, 1, H), lambda b, be: (be[b], 0, 0)),
          pl.BlockSpec((1, H, D), lambda b, be: (be[b], 0, 0)),
          pl.BlockSpec((1, 1, D), lambda b, be: (be[b], 0, 0)),
      ],
      out_specs=pl.BlockSpec((BB, D), lambda b, be: (b, 0)),
  )
  return pl.pallas_call(
      _ffn_body,
      grid_spec=grid_spec,
      out_shape=jax.ShapeDtypeStruct((PB, D), jnp.float32),
  )(block_expert, x_sorted, gate_padded.reshape(PB, 1),
    W1.astype(jnp.bfloat16), b1.reshape(NUM_EXPERTS, 1, H),
    W2.astype(jnp.bfloat16), b2.reshape(NUM_EXPERTS, 1, D))


def _combine_body(y_hbm, tok_hbm, out, tidx, rows, zbuf, shared, sem):
  c = lax.axis_index("c")
  s = lax.axis_index("s")
  w = s * NC + c

  zeros16 = jnp.zeros((LANES,), jnp.float32)

  def zero_body(i, _):
    def zcol(j, _):
      zbuf[i, pl.ds(j * LANES, LANES)] = zeros16
      return 0
    lax.fori_loop(0, D // LANES, zcol, 0)
    return 0

  lax.fori_loop(0, ZROWS, zero_body, 0)
  stripe = NPAD2 // NS
  for j in range(stripe // ZROWS):
    pltpu.sync_copy(zbuf, shared.at[pl.ds(s * stripe + j * ZROWS, ZROWS)])
  plsc.subcore_barrier()

  def body(i, _):
    r0 = w * GROWS + i * GB
    pltpu.sync_copy(y_hbm.at[pl.ds(r0, GB)], rows)
    pltpu.sync_copy(tok_hbm.at[pl.ds(r0, GB)], tidx)
    pltpu.sync_copy(rows, shared.at[tidx], add=True)
    return 0

  lax.fori_loop(0, GROWS // GB, body, 0)
  plsc.subcore_barrier()

  for j in range(stripe // ZROWS):
    r0 = s * stripe + j * ZROWS
    pltpu.sync_copy(shared.at[pl.ds(r0, ZROWS)], out.at[c, pl.ds(r0, ZROWS)])


def _combine(y_sorted, tok_padded):
  k = pl.kernel(
      _combine_body,
      out_type=jax.ShapeDtypeStruct((NC, NPAD2, D), jnp.float32),
      mesh=_mesh(),
      scratch_types=[
          pltpu.VMEM((GB,), jnp.int32),
          pltpu.VMEM((GB, D), jnp.float32),
          pltpu.VMEM((ZROWS, D), jnp.float32),
          pltpu.MemorySpace.VMEM_SHARED((NPAD2, D), jnp.float32),
          pltpu.SemaphoreType.DMA,
      ],
  )
  return k(y_sorted, tok_padded)


def _bn_body(h_ref, moe_ref, gamma_ref, beta_ref, out_ref):
  z = h_ref[...] + moe_ref[0, 0:N] + moe_ref[1, 0:N]
  mean = jnp.mean(z, axis=0, keepdims=True)
  zc = z - mean
  var = jnp.mean(zc * zc, axis=0, keepdims=True)
  out_ref[...] = zc * lax.rsqrt(var + 1e-5) * gamma_ref[...][None, :] \
      + beta_ref[...][None, :]


def _batchnorm(h, moe_parts, gamma, beta):
  return pl.pallas_call(
      _bn_body,
      out_shape=jax.ShapeDtypeStruct((N, D), jnp.float32),
  )(h, moe_parts, gamma, beta)


def kernel(feats, edge_index, W_conv, b_conv, gate_W, gate_b, W1, b1, W2, b2,
           bn_gamma, bn_beta):
  eflat = edge_index.reshape(2 * E)
  cnts = _degrees(eflat)
  hpre, isr_in = _scale(cnts, feats)
  parts = _aggregate(hpre, eflat)
  h, idx2, gates = _conv_gate(parts, isr_in, W_conv, b_conv, gate_W, gate_b)

  ex = idx2.reshape(A2)
  gflat = gates.reshape(A2)
  perm = jnp.argsort(ex, stable=True).astype(jnp.int32)
  ex_sorted = jnp.take(ex, perm)
  offsets = jnp.searchsorted(ex_sorted, jnp.arange(NUM_EXPERTS, dtype=jnp.int32),
                             side="left").astype(jnp.int32)
  counts = jnp.diff(jnp.append(offsets, A2)).astype(jnp.int32)
  pcounts = ((counts + BB - 1) // BB) * BB
  poffsets = jnp.concatenate([jnp.zeros((1,), jnp.int32),
                              jnp.cumsum(pcounts)]).astype(jnp.int32)
  block_expert = jnp.clip(
      jnp.searchsorted(poffsets, jnp.arange(NB, dtype=jnp.int32) * BB,
                       side="right").astype(jnp.int32) - 1, 0, NUM_EXPERTS - 1)
  p = jnp.arange(PB, dtype=jnp.int32)
  r = p - jnp.repeat(jnp.take(poffsets, block_expert), BB)
  valid = r < jnp.repeat(jnp.take(counts, block_expert), BB)
  j = jnp.minimum(jnp.repeat(jnp.take(offsets, block_expert), BB) + r, A2 - 1)
  a = jnp.take(perm, j)
  tok_padded = jnp.where(valid, a // TOP_K, p & 8191)
  gate_padded = jnp.where(valid, jnp.take(gflat, a), 0.0)

  x_sorted = _gather_rows(h, tok_padded)
  y_sorted = _ffn(x_sorted, gate_padded, block_expert, W1, b1, W2, b2)
  moe_parts = _combine(y_sorted, tok_padded)
  return _batchnorm(h, moe_parts, bn_gamma, bn_beta)


# speedup vs baseline: 2.0794x; 1.2109x over previous
import functools

import jax
import jax.numpy as jnp
from jax import lax
from jax.experimental import pallas as pl
from jax.experimental.pallas import tpu as pltpu
from jax.experimental.pallas import tpu_sc as plsc

N = 10000
E = 320000
D = 128
H = 256
NUM_EXPERTS = 64
TOP_K = 2

NC = 2
NS = 16
LANES = 16
NW = NC * NS

EDGES_PER_TILE = E // NW
EDGE_BATCH = 128
NEB = E // EDGE_BATCH
NEB_MAIN = NEB // NW
NPAD = 10240
ROWS_PER_TILE = NPAD // NS
ZROWS = 64


def _mesh():
  return plsc.VectorSubcoreMesh(
      core_axis_name="c", subcore_axis_name="s", num_cores=NC,
      num_subcores=NS)


def _degrees_body(eflat, out, sidx, didx, ones_v, zbuf,
                  cnt_out_sh, cnt_in_sh):
  c = lax.axis_index("c")
  s = lax.axis_index("s")
  w = s * NC + c

  zeros16 = jnp.zeros((LANES,), jnp.float32)

  def zero_body(i, _):
    zbuf[pl.ds(i * LANES, LANES)] = zeros16
    return 0

  lax.fori_loop(0, N // LANES, zero_body, 0)

  @pl.when(s == 0)
  def _():
    pltpu.sync_copy(zbuf, cnt_out_sh)

  @pl.when(s == 1)
  def _():
    pltpu.sync_copy(zbuf, cnt_in_sh)

  def ones_body(i, _):
    ones_v[pl.ds(i * LANES, LANES)] = jnp.ones((LANES,), jnp.float32)
    return 0

  lax.fori_loop(0, EDGE_BATCH // LANES, ones_body, 0)
  plsc.subcore_barrier()

  def count_batch(b):
    base = b * EDGE_BATCH
    pltpu.sync_copy(eflat.at[pl.ds(base, EDGE_BATCH)], sidx)
    pltpu.sync_copy(eflat.at[pl.ds(E + base, EDGE_BATCH)], didx)
    pltpu.sync_copy(ones_v, cnt_out_sh.at[sidx], add=True)
    pltpu.sync_copy(ones_v, cnt_in_sh.at[didx], add=True)

  def count_body(i, _):
    count_batch(w * NEB_MAIN + i)
    return 0

  lax.fori_loop(0, NEB_MAIN, count_body, 0)

  @pl.when(w < NEB - NW * NEB_MAIN)
  def _():
    count_batch(NW * NEB_MAIN + w)

  plsc.subcore_barrier()

  @pl.when(s == 0)
  def _():
    pltpu.sync_copy(cnt_out_sh, out.at[c, 0])

  @pl.when(s == 1)
  def _():
    pltpu.sync_copy(cnt_in_sh, out.at[c, 1])


def _degrees(eflat):
  k = pl.kernel(
      _degrees_body,
      out_type=jax.ShapeDtypeStruct((NC, 2, N), jnp.float32),
      mesh=_mesh(),
      scratch_types=[
          pltpu.VMEM((EDGE_BATCH,), jnp.int32),
          pltpu.VMEM((EDGE_BATCH,), jnp.int32),
          pltpu.VMEM((EDGE_BATCH,), jnp.float32),
          pltpu.VMEM((N,), jnp.float32),
          pltpu.MemorySpace.VMEM_SHARED((N,), jnp.float32),
          pltpu.MemorySpace.VMEM_SHARED((N,), jnp.float32),
      ],
  )
  return k(eflat)


def _scale_body(cnts_ref, feats_ref, hpre_ref, isr_in_ref):
  cnts = jnp.sum(cnts_ref[...], axis=0)
  deg = jnp.maximum(cnts, 1.0)
  isr = lax.rsqrt(deg)
  hpre_ref[...] = feats_ref[...] * isr[0][:, None]
  isr_in_ref[...] = isr[1][:, None]


def _scale(cnts, feats):
  return pl.pallas_call(
      _scale_body,
      out_shape=(
          jax.ShapeDtypeStruct((N, D), jnp.float32),
          jax.ShapeDtypeStruct((N, 1), jnp.float32),
      ),
  )(cnts, feats)


def _agg_body(hpre, eflat, out, sidx_a, didx_a, rows_a, sidx_b, didx_b,
              rows_b, zbuf, shared, sem_a, sem_b):
  c = lax.axis_index("c")
  s = lax.axis_index("s")
  w = s * NC + c

  zeros16 = jnp.zeros((LANES,), jnp.float32)

  def zero_body(i, _):
    def zcol(j, _):
      zbuf[i, pl.ds(j * LANES, LANES)] = zeros16
      return 0
    lax.fori_loop(0, D // LANES, zcol, 0)
    return 0

  lax.fori_loop(0, ZROWS, zero_body, 0)
  for j in range(ROWS_PER_TILE // ZROWS):
    pltpu.sync_copy(zbuf, shared.at[pl.ds(s * ROWS_PER_TILE + j * ZROWS, ZROWS)])
  plsc.subcore_barrier()

  def fetch(b, sidx, didx, rows, sem):
    base = b * EDGE_BATCH
    pltpu.sync_copy(eflat.at[pl.ds(base, EDGE_BATCH)], sidx)
    pltpu.sync_copy(eflat.at[pl.ds(E + base, EDGE_BATCH)], didx)
    pltpu.async_copy(hpre.at[sidx], rows, sem)

  def drain_scatter(sidx, didx, rows, sem):
    pltpu.make_async_copy(hpre.at[sidx], rows, sem).wait()
    pltpu.sync_copy(rows, shared.at[didx], add=True)

  b0 = w * NEB_MAIN
  fetch(b0, sidx_a, didx_a, rows_a, sem_a)

  def edge_body(k, _):
    b = b0 + 2 * k
    fetch(b + 1, sidx_b, didx_b, rows_b, sem_b)
    drain_scatter(sidx_a, didx_a, rows_a, sem_a)

    @pl.when(k < NEB_MAIN // 2 - 1)
    def _():
      fetch(b + 2, sidx_a, didx_a, rows_a, sem_a)

    drain_scatter(sidx_b, didx_b, rows_b, sem_b)
    return 0

  lax.fori_loop(0, NEB_MAIN // 2, edge_body, 0)

  @pl.when(w < NEB - NW * NEB_MAIN)
  def _():
    fetch(NW * NEB_MAIN + w, sidx_a, didx_a, rows_a, sem_a)
    drain_scatter(sidx_a, didx_a, rows_a, sem_a)

  plsc.subcore_barrier()

  for j in range(ROWS_PER_TILE // ZROWS):
    r0 = s * ROWS_PER_TILE + j * ZROWS
    pltpu.sync_copy(shared.at[pl.ds(r0, ZROWS)], out.at[c, pl.ds(r0, ZROWS)])


def _aggregate(hpre, eflat):
  k = pl.kernel(
      _agg_body,
      out_type=jax.ShapeDtypeStruct((NC, NPAD, D), jnp.float32),
      mesh=_mesh(),
      scratch_types=[
          pltpu.VMEM((EDGE_BATCH,), jnp.int32),
          pltpu.VMEM((EDGE_BATCH,), jnp.int32),
          pltpu.VMEM((EDGE_BATCH, D), jnp.float32),
          pltpu.VMEM((EDGE_BATCH,), jnp.int32),
          pltpu.VMEM((EDGE_BATCH,), jnp.int32),
          pltpu.VMEM((EDGE_BATCH, D), jnp.float32),
          pltpu.VMEM((ZROWS, D), jnp.float32),
          pltpu.MemorySpace.VMEM_SHARED((NPAD, D), jnp.float32),
          pltpu.SemaphoreType.DMA,
          pltpu.SemaphoreType.DMA,
      ],
  )
  return k(hpre, eflat)


def _conv_gate_body(p_ref, isr_ref, wc_ref, bc_ref, gw_ref, gb_ref,
                    h_ref, idx_ref, g_ref):
  agg = (p_ref[0, 0:N] + p_ref[1, 0:N]) * isr_ref[...]
  h = jnp.dot(agg, wc_ref[...], preferred_element_type=jnp.float32)
  h = h + bc_ref[...][None, :]
  h_ref[...] = h
  logits = jnp.dot(h, gw_ref[...], preferred_element_type=jnp.float32)
  logits = logits + gb_ref[...][None, :]
  iota = lax.broadcasted_iota(jnp.int32, (N, NUM_EXPERTS), 1)
  m1 = jnp.max(logits, axis=1, keepdims=True)
  i1 = jnp.min(jnp.where(logits == m1, iota, NUM_EXPERTS), axis=1,
               keepdims=True)
  masked = jnp.where(iota == i1, -jnp.inf, logits)
  m2 = jnp.max(masked, axis=1, keepdims=True)
  i2 = jnp.min(jnp.where(masked == m2, iota, NUM_EXPERTS), axis=1,
               keepdims=True)
  e2 = jnp.exp(m2 - m1)
  denom = 1.0 + e2
  g1 = 1.0 / denom
  g2 = e2 / denom
  idx_ref[...] = jnp.concatenate([i1, i2], axis=1)
  g_ref[...] = jnp.concatenate([g1, g2], axis=1)


def _conv_gate(parts, isr_in, W_conv, b_conv, gate_W, gate_b):
  return pl.pallas_call(
      _conv_gate_body,
      out_shape=(
          jax.ShapeDtypeStruct((N, D), jnp.float32),
          jax.ShapeDtypeStruct((N, TOP_K), jnp.int32),
          jax.ShapeDtypeStruct((N, TOP_K), jnp.float32),
      ),
  )(parts, isr_in, W_conv, b_conv, gate_W, gate_b)


A2 = N * TOP_K
BB = 256
PB = 36864
NB = PB // BB
SPREAD = 8192
GB = 128
GROWS = PB // NW
NPAD2 = 10240
CTOK = NPAD2 // NW
CB = 40


def _gather_rows_body(h_hbm, tok_hbm, out, tidx, rows, sem):
  c = lax.axis_index("c")
  s = lax.axis_index("s")
  w = s * NC + c
  base = w * GROWS

  def body(i, _):
    r0 = base + i * GB
    pltpu.sync_copy(tok_hbm.at[pl.ds(r0, GB)], tidx)
    pltpu.async_copy(h_hbm.at[tidx], rows, sem).wait()
    pltpu.sync_copy(rows, out.at[pl.ds(r0, GB)])
    return 0

  lax.fori_loop(0, GROWS // GB, body, 0)


def _gather_rows(h, tok_padded):
  k = pl.kernel(
      _gather_rows_body,
      out_type=jax.ShapeDtypeStruct((PB, D), jnp.float32),
      mesh=_mesh(),
      scratch_types=[
          pltpu.VMEM((GB,), jnp.int32),
          pltpu.VMEM((GB, D), jnp.float32),
          pltpu.SemaphoreType.DMA,
      ],
  )
  return k(h, tok_padded)


def _route_meta_body(exs_ref, j_ref, be_ref):
  exs = exs_ref[...]
  erange = lax.broadcasted_iota(jnp.int32, (A2, NUM_EXPERTS), 1)
  onehot = (exs[:, None] == erange).astype(jnp.int32)
  counts = jnp.sum(onehot, axis=0)
  pcounts = ((counts + BB - 1) // BB) * BB
  tri = (lax.broadcasted_iota(jnp.int32, (NUM_EXPERTS, NUM_EXPERTS), 0)
         <= lax.broadcasted_iota(jnp.int32, (NUM_EXPERTS, NUM_EXPERTS), 1))
  cum = jnp.sum(jnp.where(tri, pcounts[:, None], 0), axis=0)
  poffsets = cum - pcounts
  offsets = jnp.sum(jnp.where(tri, counts[:, None], 0), axis=0) - counts
  bvals = lax.broadcasted_iota(jnp.int32, (NB,), 0) * BB
  e_blk = jnp.minimum(
      jnp.sum((bvals[:, None] >= cum[None, :]).astype(jnp.int32), axis=1),
      NUM_EXPERTS - 1)
  bh = (e_blk[:, None] == lax.broadcasted_iota(jnp.int32, (NB, NUM_EXPERTS), 1)
        ).astype(jnp.int32)
  poff_blk = jnp.sum(bh * poffsets[None, :], axis=1)
  off_blk = jnp.sum(bh * offsets[None, :], axis=1)
  cnt_blk = jnp.sum(bh * counts[None, :], axis=1)
  i_in = lax.broadcasted_iota(jnp.int32, (NB, BB), 1)
  r = bvals[:, None] + i_in - poff_blk[:, None]
  valid = r < cnt_blk[:, None]
  p_all = bvals[:, None] + i_in
  j_ref[...] = jnp.where(valid, off_blk[:, None] + r,
                         A2 + (p_all & (SPREAD - 1)))
  be_ref[...] = e_blk


def _route_meta(ex_sorted):
  return pl.pallas_call(
      _route_meta_body,
      out_shape=(
          jax.ShapeDtypeStruct((NB, BB), jnp.int32),
          jax.ShapeDtypeStruct((NB,), jnp.int32),
      ),
  )(ex_sorted)


def _ffn_body(be_ref, x_ref, g_ref, w1_ref, b1_ref, w2_ref, b2_ref, y_ref):
  x = x_ref[...]
  a = jnp.dot(x.astype(jnp.bfloat16), w1_ref[0],
              preferred_element_type=jnp.float32)
  a = a + b1_ref[0]
  a = 0.5 * a * (1.0 + lax.erf(a * 0.7071067811865476))
  y = jnp.dot(a.astype(jnp.bfloat16), w2_ref[0],
              preferred_element_type=jnp.float32)
  y = y + b2_ref[0]
  y_ref[...] = y * g_ref[...]


def _ffn(x_sorted, gate_padded, block_expert, W1, b1, W2, b2):
  grid_spec = pltpu.PrefetchScalarGridSpec(
      num_scalar_prefetch=1,
      grid=(NB,),
      in_specs=[
          pl.BlockSpec((BB, D), lambda b, be: (b, 0)),
          pl.BlockSpec((BB, 1), lambda b, be: (b, 0)),
          pl.BlockSpec((1, D, H), lambda b, be: (be[b], 0, 0)),
          pl.BlockSpec((1, 1, H), lambda b, be: (be[b], 0, 0)),
          pl.BlockSpec((1, H, D), lambda b, be: (be[b], 0, 0)),
          pl.BlockSpec((1, 1, D), lambda b, be: (be[b], 0, 0)),
      ],
      out_specs=pl.BlockSpec((BB, D), lambda b, be: (b, 0)),
  )
  return pl.pallas_call(
      _ffn_body,
      grid_spec=grid_spec,
      out_shape=jax.ShapeDtypeStruct((PB, D), jnp.float32),
  )(block_expert, x_sorted, gate_padded.reshape(PB, 1),
    W1.astype(jnp.bfloat16), b1.reshape(NUM_EXPERTS, 1, H),
    W2.astype(jnp.bfloat16), b2.reshape(NUM_EXPERTS, 1, D))


def _combine_body(y_hbm, tok_hbm, out, tidx, rows, zbuf, shared, sem):
  c = lax.axis_index("c")
  s = lax.axis_index("s")
  w = s * NC + c

  zeros16 = jnp.zeros((LANES,), jnp.float32)

  def zero_body(i, _):
    def zcol(j, _):
      zbuf[i, pl.ds(j * LANES, LANES)] = zeros16
      return 0
    lax.fori_loop(0, D // LANES, zcol, 0)
    return 0

  lax.fori_loop(0, ZROWS, zero_body, 0)
  stripe = NPAD2 // NS
  for j in range(stripe // ZROWS):
    pltpu.sync_copy(zbuf, shared.at[pl.ds(s * stripe + j * ZROWS, ZROWS)])
  plsc.subcore_barrier()

  def body(i, _):
    r0 = w * GROWS + i * GB
    pltpu.sync_copy(y_hbm.at[pl.ds(r0, GB)], rows)
    pltpu.sync_copy(tok_hbm.at[pl.ds(r0, GB)], tidx)
    pltpu.sync_copy(rows, shared.at[tidx], add=True)
    return 0

  lax.fori_loop(0, GROWS // GB, body, 0)
  plsc.subcore_barrier()

  for j in range(stripe // ZROWS):
    r0 = s * stripe + j * ZROWS
    pltpu.sync_copy(shared.at[pl.ds(r0, ZROWS)], out.at[c, pl.ds(r0, ZROWS)])


def _combine(y_sorted, tok_padded):
  k = pl.kernel(
      _combine_body,
      out_type=jax.ShapeDtypeStruct((NC, NPAD2, D), jnp.float32),
      mesh=_mesh(),
      scratch_types=[
          pltpu.VMEM((GB,), jnp.int32),
          pltpu.VMEM((GB, D), jnp.float32),
          pltpu.VMEM((ZROWS, D), jnp.float32),
          pltpu.MemorySpace.VMEM_SHARED((NPAD2, D), jnp.float32),
          pltpu.SemaphoreType.DMA,
      ],
  )
  return k(y_sorted, tok_padded)


def _bn_body(h_ref, moe_ref, gamma_ref, beta_ref, out_ref):
  z = h_ref[...] + moe_ref[0, 0:N] + moe_ref[1, 0:N]
  mean = jnp.mean(z, axis=0, keepdims=True)
  zc = z - mean
  var = jnp.mean(zc * zc, axis=0, keepdims=True)
  out_ref[...] = zc * lax.rsqrt(var + 1e-5) * gamma_ref[...][None, :] \
      + beta_ref[...][None, :]


def _batchnorm(h, moe_parts, gamma, beta):
  return pl.pallas_call(
      _bn_body,
      out_shape=jax.ShapeDtypeStruct((N, D), jnp.float32),
  )(h, moe_parts, gamma, beta)


def kernel(feats, edge_index, W_conv, b_conv, gate_W, gate_b, W1, b1, W2, b2,
           bn_gamma, bn_beta):
  eflat = edge_index.reshape(2 * E)
  cnts = _degrees(eflat)
  hpre, isr_in = _scale(cnts, feats)
  parts = _aggregate(hpre, eflat)
  h, idx2, gates = _conv_gate(parts, isr_in, W_conv, b_conv, gate_W, gate_b)

  ex = idx2.reshape(A2)
  gflat = gates.reshape(A2)
  perm = jnp.argsort(ex, stable=True).astype(jnp.int32)
  ex_sorted = jnp.take(ex, perm)
  j2, block_expert = _route_meta(ex_sorted)
  ptok_ext = jnp.concatenate(
      [perm // TOP_K, jnp.arange(SPREAD, dtype=jnp.int32)])
  gfs_ext = jnp.concatenate(
      [jnp.take(gflat, perm), jnp.zeros((SPREAD,), jnp.float32)])
  jf = j2.reshape(PB)
  tok_padded = jnp.take(ptok_ext, jf)
  gate_padded = jnp.take(gfs_ext, jf)

  x_sorted = _gather_rows(h, tok_padded)
  y_sorted = _ffn(x_sorted, gate_padded, block_expert, W1, b1, W2, b2)
  moe_parts = _combine(y_sorted, tok_padded)
  return _batchnorm(h, moe_parts, bn_gamma, bn_beta)


# speedup vs baseline: 2.4084x; 1.1582x over previous
import functools

import jax
import jax.numpy as jnp
from jax import lax
from jax.experimental import pallas as pl
from jax.experimental.pallas import tpu as pltpu
from jax.experimental.pallas import tpu_sc as plsc

N = 10000
E = 320000
D = 128
H = 256
NUM_EXPERTS = 64
TOP_K = 2

NC = 2
NS = 16
LANES = 16
NW = NC * NS

EDGES_PER_TILE = E // NW
EDGE_BATCH = 128
NEB = E // EDGE_BATCH
TB_FULL = 80
TB_LAST = NEB - 31 * TB_FULL
NPAD = 10240
ROWS_PER_TILE = NPAD // NS
ZROWS = 32


def _mesh():
  return plsc.VectorSubcoreMesh(
      core_axis_name="c", subcore_axis_name="s", num_cores=NC,
      num_subcores=NS)


def _degrees_body(e2d, eflat, out, sidx2d, didx2d, sidx1, didx1, ones_v, zbuf,
                  cnt_out_sh, cnt_in_sh):
  c = lax.axis_index("c")
  s = lax.axis_index("s")
  w = s * NC + c

  zeros16 = jnp.zeros((LANES,), jnp.float32)

  def zero_body(i, _):
    zbuf[pl.ds(i * LANES, LANES)] = zeros16
    return 0

  lax.fori_loop(0, N // LANES, zero_body, 0)

  @pl.when(s == 0)
  def _():
    pltpu.sync_copy(zbuf, cnt_out_sh)

  @pl.when(s == 1)
  def _():
    pltpu.sync_copy(zbuf, cnt_in_sh)

  def ones_body(i, _):
    ones_v[pl.ds(i * LANES, LANES)] = jnp.ones((LANES,), jnp.float32)
    return 0

  lax.fori_loop(0, EDGE_BATCH // LANES, ones_body, 0)
  plsc.subcore_barrier()

  @pl.when(w < NW - 1)
  def _():
    pltpu.sync_copy(e2d.at[0, pl.ds(w * TB_FULL, TB_FULL)], sidx2d)
    pltpu.sync_copy(e2d.at[1, pl.ds(w * TB_FULL, TB_FULL)], didx2d)

    def count_body(i, _):
      pltpu.sync_copy(ones_v, cnt_out_sh.at[sidx2d.at[i]], add=True)
      pltpu.sync_copy(ones_v, cnt_in_sh.at[didx2d.at[i]], add=True)
      return 0

    lax.fori_loop(0, TB_FULL, count_body, 0)

  @pl.when(w == NW - 1)
  def _():
    def count_tail(i, _):
      base = (w * TB_FULL + i) * EDGE_BATCH
      pltpu.sync_copy(eflat.at[pl.ds(base, EDGE_BATCH)], sidx1)
      pltpu.sync_copy(eflat.at[pl.ds(E + base, EDGE_BATCH)], didx1)
      pltpu.sync_copy(ones_v, cnt_out_sh.at[sidx1], add=True)
      pltpu.sync_copy(ones_v, cnt_in_sh.at[didx1], add=True)
      return 0

    lax.fori_loop(0, TB_LAST, count_tail, 0)

  plsc.subcore_barrier()

  @pl.when(s == 0)
  def _():
    pltpu.sync_copy(cnt_out_sh, out.at[c, 0])

  @pl.when(s == 1)
  def _():
    pltpu.sync_copy(cnt_in_sh, out.at[c, 1])


def _degrees(e2d, eflat):
  k = pl.kernel(
      _degrees_body,
      out_type=jax.ShapeDtypeStruct((NC, 2, N), jnp.float32),
      mesh=_mesh(),
      scratch_types=[
          pltpu.VMEM((TB_FULL, EDGE_BATCH), jnp.int32),
          pltpu.VMEM((TB_FULL, EDGE_BATCH), jnp.int32),
          pltpu.VMEM((EDGE_BATCH,), jnp.int32),
          pltpu.VMEM((EDGE_BATCH,), jnp.int32),
          pltpu.VMEM((EDGE_BATCH,), jnp.float32),
          pltpu.VMEM((N,), jnp.float32),
          pltpu.MemorySpace.VMEM_SHARED((N,), jnp.float32),
          pltpu.MemorySpace.VMEM_SHARED((N,), jnp.float32),
      ],
  )
  return k(e2d, eflat)


def _scale_body(cnts_ref, feats_ref, hpre_ref, isr_in_ref):
  cnts = jnp.sum(cnts_ref[...], axis=0)
  deg = jnp.maximum(cnts, 1.0)
  isr = lax.rsqrt(deg)
  hpre_ref[...] = feats_ref[...] * isr[0][:, None]
  isr_in_ref[...] = isr[1][:, None]


def _scale(cnts, feats):
  return pl.pallas_call(
      _scale_body,
      out_shape=(
          jax.ShapeDtypeStruct((N, D), jnp.float32),
          jax.ShapeDtypeStruct((N, 1), jnp.float32),
      ),
  )(cnts, feats)


def _agg_body(hpre, e2d, eflat, out, sidx_a, rows_a, sidx_b, rows_b, didx2d,
              didx1, zbuf, shared, sem_a, sem_b):
  c = lax.axis_index("c")
  s = lax.axis_index("s")
  w = s * NC + c

  zeros16 = jnp.zeros((LANES,), jnp.float32)

  def zero_body(i, _):
    def zcol(j, _):
      zbuf[i, pl.ds(j * LANES, LANES)] = zeros16
      return 0
    lax.fori_loop(0, D // LANES, zcol, 0)
    return 0

  lax.fori_loop(0, ZROWS, zero_body, 0)
  for j in range(ROWS_PER_TILE // ZROWS):
    pltpu.sync_copy(zbuf, shared.at[pl.ds(s * ROWS_PER_TILE + j * ZROWS, ZROWS)])

  def fetch(i, sidx, rows, sem):
    pltpu.sync_copy(eflat.at[pl.ds((w * TB_FULL + i) * EDGE_BATCH, EDGE_BATCH)],
                    sidx)
    pltpu.async_copy(hpre.at[sidx], rows, sem)

  @pl.when(w < NW - 1)
  def _():
    pltpu.sync_copy(e2d.at[1, pl.ds(w * TB_FULL, TB_FULL)], didx2d)

    def drain_scatter(i, sidx, rows, sem):
      pltpu.make_async_copy(hpre.at[sidx], rows, sem).wait()
      pltpu.sync_copy(rows, shared.at[didx2d.at[i]], add=True)

    fetch(0, sidx_a, rows_a, sem_a)

    def edge_body(k, _):
      i = 2 * k
      fetch(i + 1, sidx_b, rows_b, sem_b)
      drain_scatter(i, sidx_a, rows_a, sem_a)

      @pl.when(k < TB_FULL // 2 - 1)
      def _():
        fetch(i + 2, sidx_a, rows_a, sem_a)

      drain_scatter(i + 1, sidx_b, rows_b, sem_b)
      return 0

    lax.fori_loop(0, TB_FULL // 2, edge_body, 0)

  @pl.when(w == NW - 1)
  def _():
    def tail_body(i, _):
      base = (w * TB_FULL + i) * EDGE_BATCH
      pltpu.sync_copy(eflat.at[pl.ds(base, EDGE_BATCH)], sidx_a)
      pltpu.sync_copy(eflat.at[pl.ds(E + base, EDGE_BATCH)], didx1)
      pltpu.async_copy(hpre.at[sidx_a], rows_a, sem_a).wait()
      pltpu.sync_copy(rows_a, shared.at[didx1], add=True)
      return 0

    lax.fori_loop(0, TB_LAST, tail_body, 0)

  plsc.subcore_barrier()

  for j in range(ROWS_PER_TILE // ZROWS):
    r0 = s * ROWS_PER_TILE + j * ZROWS
    pltpu.sync_copy(shared.at[pl.ds(r0, ZROWS)], out.at[c, pl.ds(r0, ZROWS)])


def _aggregate(hpre, e2d, eflat):
  k = pl.kernel(
      _agg_body,
      out_type=jax.ShapeDtypeStruct((NC, NPAD, D), jnp.float32),
      mesh=_mesh(),
      scratch_types=[
          pltpu.VMEM((EDGE_BATCH,), jnp.int32),
          pltpu.VMEM((EDGE_BATCH, D), jnp.float32),
          pltpu.VMEM((EDGE_BATCH,), jnp.int32),
          pltpu.VMEM((EDGE_BATCH, D), jnp.float32),
          pltpu.VMEM((TB_FULL, EDGE_BATCH), jnp.int32),
          pltpu.VMEM((EDGE_BATCH,), jnp.int32),
          pltpu.VMEM((ZROWS, D), jnp.float32),
          pltpu.MemorySpace.VMEM_SHARED((NPAD, D), jnp.float32),
          pltpu.SemaphoreType.DMA,
          pltpu.SemaphoreType.DMA,
      ],
  )
  return k(hpre, e2d, eflat)


def _conv_gate_body(p_ref, isr_ref, wc_ref, bc_ref, gw_ref, gb_ref,
                    h_ref, idx_ref, g_ref):
  agg = (p_ref[0, 0:N] + p_ref[1, 0:N]) * isr_ref[...]
  h = jnp.dot(agg, wc_ref[...], preferred_element_type=jnp.float32)
  h = h + bc_ref[...][None, :]
  h_ref[...] = h
  logits = jnp.dot(h, gw_ref[...], preferred_element_type=jnp.float32)
  logits = logits + gb_ref[...][None, :]
  iota = lax.broadcasted_iota(jnp.int32, (N, NUM_EXPERTS), 1)
  m1 = jnp.max(logits, axis=1, keepdims=True)
  i1 = jnp.min(jnp.where(logits == m1, iota, NUM_EXPERTS), axis=1,
               keepdims=True)
  masked = jnp.where(iota == i1, -jnp.inf, logits)
  m2 = jnp.max(masked, axis=1, keepdims=True)
  i2 = jnp.min(jnp.where(masked == m2, iota, NUM_EXPERTS), axis=1,
               keepdims=True)
  e2 = jnp.exp(m2 - m1)
  denom = 1.0 + e2
  g1 = 1.0 / denom
  g2 = e2 / denom
  idx_ref[...] = jnp.concatenate([i1, i2], axis=1)
  g_ref[...] = jnp.concatenate([g1, g2], axis=1)


def _conv_gate(parts, isr_in, W_conv, b_conv, gate_W, gate_b):
  return pl.pallas_call(
      _conv_gate_body,
      out_shape=(
          jax.ShapeDtypeStruct((N, D), jnp.float32),
          jax.ShapeDtypeStruct((N, TOP_K), jnp.int32),
          jax.ShapeDtypeStruct((N, TOP_K), jnp.float32),
      ),
  )(parts, isr_in, W_conv, b_conv, gate_W, gate_b)


A2 = N * TOP_K
BB = 256
PB = 36864
NB = PB // BB
SPREAD = 8192
GB = 128
GROWS = PB // NW
NPAD2 = 10240
CTOK = NPAD2 // NW
CB = 40


def _gather_rows_body(h_hbm, tok_hbm, out, tidx, rows, sem):
  c = lax.axis_index("c")
  s = lax.axis_index("s")
  w = s * NC + c
  base = w * GROWS

  def body(i, _):
    r0 = base + i * GB
    pltpu.sync_copy(tok_hbm.at[pl.ds(r0, GB)], tidx)
    pltpu.async_copy(h_hbm.at[tidx], rows, sem).wait()
    pltpu.sync_copy(rows, out.at[pl.ds(r0, GB)])
    return 0

  lax.fori_loop(0, GROWS // GB, body, 0)


def _gather_rows(h, tok_padded):
  k = pl.kernel(
      _gather_rows_body,
      out_type=jax.ShapeDtypeStruct((PB, D), jnp.float32),
      mesh=_mesh(),
      scratch_types=[
          pltpu.VMEM((GB,), jnp.int32),
          pltpu.VMEM((GB, D), jnp.float32),
          pltpu.SemaphoreType.DMA,
      ],
  )
  return k(h, tok_padded)


def _route_meta_body(exs_ref, j_ref, be_ref):
  exs = exs_ref[...]
  erange = lax.broadcasted_iota(jnp.int32, (A2, NUM_EXPERTS), 1)
  onehot = (exs[:, None] == erange).astype(jnp.int32)
  counts = jnp.sum(onehot, axis=0)
  pcounts = ((counts + BB - 1) // BB) * BB
  tri = (lax.broadcasted_iota(jnp.int32, (NUM_EXPERTS, NUM_EXPERTS), 0)
         <= lax.broadcasted_iota(jnp.int32, (NUM_EXPERTS, NUM_EXPERTS), 1))
  cum = jnp.sum(jnp.where(tri, pcounts[:, None], 0), axis=0)
  poffsets = cum - pcounts
  offsets = jnp.sum(jnp.where(tri, counts[:, None], 0), axis=0) - counts
  bvals = lax.broadcasted_iota(jnp.int32, (NB,), 0) * BB
  e_blk = jnp.minimum(
      jnp.sum((bvals[:, None] >= cum[None, :]).astype(jnp.int32), axis=1),
      NUM_EXPERTS - 1)
  bh = (e_blk[:, None] == lax.broadcasted_iota(jnp.int32, (NB, NUM_EXPERTS), 1)
        ).astype(jnp.int32)
  poff_blk = jnp.sum(bh * poffsets[None, :], axis=1)
  off_blk = jnp.sum(bh * offsets[None, :], axis=1)
  cnt_blk = jnp.sum(bh * counts[None, :], axis=1)
  i_in = lax.broadcasted_iota(jnp.int32, (NB, BB), 1)
  r = bvals[:, None] + i_in - poff_blk[:, None]
  valid = r < cnt_blk[:, None]
  p_all = bvals[:, None] + i_in
  j_ref[...] = jnp.where(valid, off_blk[:, None] + r,
                         A2 + (p_all & (SPREAD - 1)))
  be_ref[...] = e_blk


def _route_meta(ex_sorted):
  return pl.pallas_call(
      _route_meta_body,
      out_shape=(
          jax.ShapeDtypeStruct((NB, BB), jnp.int32),
          jax.ShapeDtypeStruct((NB,), jnp.int32),
      ),
  )(ex_sorted)


def _ffn_body(be_ref, x_ref, g_ref, w1_ref, b1_ref, w2_ref, b2_ref, y_ref):
  x = x_ref[...]
  a = jnp.dot(x.astype(jnp.bfloat16), w1_ref[0],
              preferred_element_type=jnp.float32)
  a = a + b1_ref[0]
  a = 0.5 * a * (1.0 + lax.erf(a * 0.7071067811865476))
  y = jnp.dot(a.astype(jnp.bfloat16), w2_ref[0],
              preferred_element_type=jnp.float32)
  y = y + b2_ref[0]
  y_ref[...] = y * g_ref[...]


def _ffn(x_sorted, gate_padded, block_expert, W1, b1, W2, b2):
  grid_spec = pltpu.PrefetchScalarGridSpec(
      num_scalar_prefetch=1,
      grid=(NB,),
      in_specs=[
          pl.BlockSpec((BB, D), lambda b, be: (b, 0)),
          pl.BlockSpec((BB, 1), lambda b, be: (b, 0)),
          pl.BlockSpec((1, D, H), lambda b, be: (be[b], 0, 0)),
          pl.BlockSpec((1, 1, H), lambda b, be: (be[b], 0, 0)),
          pl.BlockSpec((1, H, D), lambda b, be: (be[b], 0, 0)),
          pl.BlockSpec((1, 1, D), lambda b, be: (be[b], 0, 0)),
      ],
      out_specs=pl.BlockSpec((BB, D), lambda b, be: (b, 0)),
  )
  return pl.pallas_call(
      _ffn_body,
      grid_spec=grid_spec,
      out_shape=jax.ShapeDtypeStruct((PB, D), jnp.float32),
  )(block_expert, x_sorted, gate_padded.reshape(PB, 1),
    W1.astype(jnp.bfloat16), b1.reshape(NUM_EXPERTS, 1, H),
    W2.astype(jnp.bfloat16), b2.reshape(NUM_EXPERTS, 1, D))


def _combine_body(y_hbm, tok_hbm, out, tidx, rows, zbuf, shared, sem):
  c = lax.axis_index("c")
  s = lax.axis_index("s")
  w = s * NC + c

  zeros16 = jnp.zeros((LANES,), jnp.float32)

  def zero_body(i, _):
    def zcol(j, _):
      zbuf[i, pl.ds(j * LANES, LANES)] = zeros16
      return 0
    lax.fori_loop(0, D // LANES, zcol, 0)
    return 0

  lax.fori_loop(0, ZROWS, zero_body, 0)
  stripe = NPAD2 // NS
  for j in range(stripe // ZROWS):
    pltpu.sync_copy(zbuf, shared.at[pl.ds(s * stripe + j * ZROWS, ZROWS)])
  plsc.subcore_barrier()

  def body(i, _):
    r0 = w * GROWS + i * GB
    pltpu.sync_copy(y_hbm.at[pl.ds(r0, GB)], rows)
    pltpu.sync_copy(tok_hbm.at[pl.ds(r0, GB)], tidx)
    pltpu.sync_copy(rows, shared.at[tidx], add=True)
    return 0

  lax.fori_loop(0, GROWS // GB, body, 0)
  plsc.subcore_barrier()

  for j in range(stripe // ZROWS):
    r0 = s * stripe + j * ZROWS
    pltpu.sync_copy(shared.at[pl.ds(r0, ZROWS)], out.at[c, pl.ds(r0, ZROWS)])


def _combine(y_sorted, tok_padded):
  k = pl.kernel(
      _combine_body,
      out_type=jax.ShapeDtypeStruct((NC, NPAD2, D), jnp.float32),
      mesh=_mesh(),
      scratch_types=[
          pltpu.VMEM((GB,), jnp.int32),
          pltpu.VMEM((GB, D), jnp.float32),
          pltpu.VMEM((ZROWS, D), jnp.float32),
          pltpu.MemorySpace.VMEM_SHARED((NPAD2, D), jnp.float32),
          pltpu.SemaphoreType.DMA,
      ],
  )
  return k(y_sorted, tok_padded)


def _bn_body(h_ref, moe_ref, gamma_ref, beta_ref, out_ref):
  z = h_ref[...] + moe_ref[0, 0:N] + moe_ref[1, 0:N]
  mean = jnp.mean(z, axis=0, keepdims=True)
  zc = z - mean
  var = jnp.mean(zc * zc, axis=0, keepdims=True)
  out_ref[...] = zc * lax.rsqrt(var + 1e-5) * gamma_ref[...][None, :] \
      + beta_ref[...][None, :]


def _batchnorm(h, moe_parts, gamma, beta):
  return pl.pallas_call(
      _bn_body,
      out_shape=jax.ShapeDtypeStruct((N, D), jnp.float32),
  )(h, moe_parts, gamma, beta)


def kernel(feats, edge_index, W_conv, b_conv, gate_W, gate_b, W1, b1, W2, b2,
           bn_gamma, bn_beta):
  e2d = edge_index.reshape(2, NEB, EDGE_BATCH)
  eflat = edge_index.reshape(2 * E)
  cnts = _degrees(e2d, eflat)
  hpre, isr_in = _scale(cnts, feats)
  parts = _aggregate(hpre, e2d, eflat)
  h, idx2, gates = _conv_gate(parts, isr_in, W_conv, b_conv, gate_W, gate_b)

  ex = idx2.reshape(A2)
  gflat = gates.reshape(A2)
  perm = jnp.argsort(ex, stable=True).astype(jnp.int32)
  ex_sorted = jnp.take(ex, perm)
  j2, block_expert = _route_meta(ex_sorted)
  ptok_ext = jnp.concatenate(
      [perm // TOP_K, jnp.arange(SPREAD, dtype=jnp.int32)])
  gfs_ext = jnp.concatenate(
      [jnp.take(gflat, perm), jnp.zeros((SPREAD,), jnp.float32)])
  jf = j2.reshape(PB)
  tok_padded = jnp.take(ptok_ext, jf)
  gate_padded = jnp.take(gfs_ext, jf)

  x_sorted = _gather_rows(h, tok_padded)
  y_sorted = _ffn(x_sorted, gate_padded, block_expert, W1, b1, W2, b2)
  moe_parts = _combine(y_sorted, tok_padded)
  return _batchnorm(h, moe_parts, bn_gamma, bn_beta)
